# baseline (speedup 1.0000x reference)
import functools
import math

import jax
import jax.numpy as jnp
import numpy as np
from jax import lax
from jax.experimental import pallas as pl
from jax.experimental.pallas import tpu as pltpu
from jax.experimental.pallas import tpu_sc as plsc

H_DIM = 768
N_HEADS = 12
C_DIM = 256
N_TOKENS = 32000
N_ROUTED = 8
TOP_K = 2
E_DIM = 4 * H_DIM
D_HEAD = H_DIM // N_HEADS
D_ROPE = D_HEAD // 2
UP_DIM = (D_HEAD - D_ROPE) * N_HEADS
SEQ = 2048

PREC = jax.lax.Precision.DEFAULT

_half = D_ROPE // 2
_inv_freq = 1.0 / (10000.0 ** (np.arange(0, _half, 2, dtype=np.float32) / _half))
_t = np.arange(SEQ, dtype=np.float32) / 40.0
_freqs = _t[:, None] * _inv_freq[None, :]
_c8 = np.cos(_freqs)
_s8 = np.sin(_freqs)
_ones16 = np.ones((SEQ, _half), np.float32)
_zeros16 = np.zeros((SEQ, _half), np.float32)
_COSF = np.tile(np.concatenate([_c8, _c8, _ones16], axis=1), (1, N_HEADS))
_SINF = np.tile(np.concatenate([_s8, _s8, _zeros16], axis=1), (1, N_HEADS))
_P = np.zeros((UP_DIM, UP_DIM), np.float32)
for _h in range(N_HEADS):
    _b = _h * D_ROPE
    for _i in range(8):
        _P[_b + 8 + _i, _b + _i] = -1.0
        _P[_b + _i, _b + 8 + _i] = 1.0
COSF = _COSF
SINF = _SINF
PMAT = _P


def _dot(a, b):
    return lax.dot_general(a, b, (((1,), (0,)), ((), ())),
                           preferred_element_type=jnp.float32, precision=PREC)


def _dot_fast(a, b):
    return lax.dot_general(a.astype(jnp.bfloat16), b.astype(jnp.bfloat16),
                           (((1,), (0,)), ((), ())),
                           preferred_element_type=jnp.float32)


def _dot_x(a, b):
    return lax.dot_general(a, b, (((1,), (0,)), ((), ())),
                           preferred_element_type=jnp.float32,
                           precision=jax.lax.Precision.HIGHEST)


def _ln(x, scale, bias):
    m = jnp.mean(x, axis=-1, keepdims=True)
    d = x - m
    v = jnp.mean(d * d, axis=-1, keepdims=True)
    return d * lax.rsqrt(v + 1e-5) * scale + bias


def _erf(x):
    ax = jnp.abs(x)
    t = 1.0 / (1.0 + 0.3275911 * ax)
    y = ((((1.061405429 * t - 1.453152027) * t + 1.421413741) * t
          - 0.284496736) * t + 0.254829592) * t
    y = 1.0 - y * jnp.exp(-ax * ax)
    return jnp.sign(x) * y


def _gelu(x):
    return 0.5 * x * (1.0 + _erf(x * (1.0 / math.sqrt(2.0))))


_NW = 32
_B_PER_W = SEQ // _NW


def _emb_gather(emb, idx):
    mesh = plsc.VectorSubcoreMesh(core_axis_name="c", subcore_axis_name="s")

    @functools.partial(
        pl.kernel, mesh=mesh,
        out_type=jax.ShapeDtypeStruct((SEQ, H_DIM), jnp.float32),
        scratch_types=[
            pltpu.VMEM((_B_PER_W,), jnp.int32),
            pltpu.VMEM((_B_PER_W, H_DIM), jnp.float32),
            pltpu.SemaphoreType.DMA,
        ],
    )
    def k(emb_hbm, idx_hbm, out_hbm, idx_v, rows_v, sem):
        wid = lax.axis_index("s") * 2 + lax.axis_index("c")
        base = wid * _B_PER_W
        pltpu.sync_copy(idx_hbm.at[pl.ds(base, _B_PER_W)], idx_v)
        pltpu.async_copy(emb_hbm.at[idx_v], rows_v, sem).wait()
        pltpu.sync_copy(rows_v, out_hbm.at[pl.ds(base, _B_PER_W)])

    return k(emb, idx)


_AB = 256
_NAB = SEQ // _AB


def _attn1_body(h_ref, n1s_ref, n1b_ref, wdkv_ref, wdq_ref, wuk_ref,
                wuv_ref, wuq_ref, wkr_ref, wqr_ref, cos_ref, sin_ref,
                p_ref, kc_ref, kr_ref, qc_ref, qr_ref, v_ref):
    h1 = _ln(h_ref[...], n1s_ref[...], n1b_ref[...])
    c_kv = _dot(h1, wdkv_ref[...])
    c_q = _dot(h1, wdq_ref[...])
    qc_ref[...] = _dot(c_q, wuq_ref[...])
    kc_ref[...] = _dot(c_kv, wuk_ref[...])
    v_ref[...] = _dot(c_kv, wuv_ref[...])
    k_r = _dot(h1, wkr_ref[...])
    q_r = _dot(c_q, wqr_ref[...])
    cos = cos_ref[...]
    sin = sin_ref[...]
    pm = p_ref[...]
    kr_ref[...] = k_r * cos + _dot_x(k_r, pm) * sin
    qr_ref[...] = q_r * cos + _dot_x(q_r, pm) * sin


def _attn2_body(kc_ref, kr_ref, qc_ref, qr_ref, v_ref, out_ref):
    kc = kc_ref[...]
    kr = kr_ref[...]
    qc = qc_ref[...]
    qr = qr_ref[...]
    v = v_ref[...]
    outs = []
    for hh in range(N_HEADS):
        s32 = slice(hh * D_ROPE, (hh + 1) * D_ROPE)
        s64 = slice(hh * D_HEAD, (hh + 1) * D_HEAD)
        k_cat = jnp.concatenate([kc[:, s32], kr[:, s32]], axis=1)
        q_cat = jnp.concatenate([qc[:, s32], qr[:, s32]], axis=1)
        s = lax.dot_general(k_cat, q_cat, (((1,), (1,)), ((), ())),
                            preferred_element_type=jnp.float32,
                            precision=PREC) * (1.0 / math.sqrt(D_HEAD))
        outs.append(_dot(s, v[:, s64]))
    out_ref[...] = jnp.concatenate(outs, axis=1)


def _attn3_body(h_ref, attn_ref, wo_ref, out_ref):
    out_ref[...] = _dot(attn_ref[...], wo_ref[...]) + h_ref[...]


def _attn_block(h, n1s, n1b, wdkv, wdq, wuk, wuv, wuq, wkr, wqr, wo):
    kc, kr, qc, qr, v = pl.pallas_call(
        _attn1_body,
        grid=(_NAB,),
        in_specs=[
            pl.BlockSpec((_AB, H_DIM), lambda r: (r, 0)),
            pl.BlockSpec((1, H_DIM), lambda r: (0, 0)),
            pl.BlockSpec((1, H_DIM), lambda r: (0, 0)),
            pl.BlockSpec((H_DIM, C_DIM), lambda r: (0, 0)),
            pl.BlockSpec((H_DIM, C_DIM), lambda r: (0, 0)),
            pl.BlockSpec((C_DIM, UP_DIM), lambda r: (0, 0)),
            pl.BlockSpec((C_DIM, H_DIM), lambda r: (0, 0)),
            pl.BlockSpec((C_DIM, UP_DIM), lambda r: (0, 0)),
            pl.BlockSpec((H_DIM, UP_DIM), lambda r: (0, 0)),
            pl.BlockSpec((C_DIM, UP_DIM), lambda r: (0, 0)),
            pl.BlockSpec((_AB, UP_DIM), lambda r: (r, 0)),
            pl.BlockSpec((_AB, UP_DIM), lambda r: (r, 0)),
            pl.BlockSpec((UP_DIM, UP_DIM), lambda r: (0, 0)),
        ],
        out_specs=[
            pl.BlockSpec((_AB, UP_DIM), lambda r: (r, 0)),
            pl.BlockSpec((_AB, UP_DIM), lambda r: (r, 0)),
            pl.BlockSpec((_AB, UP_DIM), lambda r: (r, 0)),
            pl.BlockSpec((_AB, UP_DIM), lambda r: (r, 0)),
            pl.BlockSpec((_AB, H_DIM), lambda r: (r, 0)),
        ],
        out_shape=[
            jax.ShapeDtypeStruct((SEQ, UP_DIM), jnp.float32),
            jax.ShapeDtypeStruct((SEQ, UP_DIM), jnp.float32),
            jax.ShapeDtypeStruct((SEQ, UP_DIM), jnp.float32),
            jax.ShapeDtypeStruct((SEQ, UP_DIM), jnp.float32),
            jax.ShapeDtypeStruct((SEQ, H_DIM), jnp.float32),
        ],
    )(h, n1s, n1b, wdkv, wdq, wuk, wuv, wuq, wkr, wqr, COSF, SINF, PMAT)
    attn = pl.pallas_call(
        _attn2_body,
        grid=(_NAB,),
        in_specs=[
            pl.BlockSpec((_AB, UP_DIM), lambda i: (i, 0)),
            pl.BlockSpec((_AB, UP_DIM), lambda i: (i, 0)),
            pl.BlockSpec((SEQ, UP_DIM), lambda i: (0, 0)),
            pl.BlockSpec((SEQ, UP_DIM), lambda i: (0, 0)),
            pl.BlockSpec((SEQ, H_DIM), lambda i: (0, 0)),
        ],
        out_specs=pl.BlockSpec((_AB, H_DIM), lambda i: (i, 0)),
        out_shape=jax.ShapeDtypeStruct((SEQ, H_DIM), jnp.float32),
    )(kc, kr, qc, qr, v)
    return pl.pallas_call(
        _attn3_body,
        grid=(_NAB,),
        in_specs=[
            pl.BlockSpec((_AB, H_DIM), lambda r: (r, 0)),
            pl.BlockSpec((_AB, H_DIM), lambda r: (r, 0)),
            pl.BlockSpec((H_DIM, H_DIM), lambda r: (0, 0)),
        ],
        out_specs=pl.BlockSpec((_AB, H_DIM), lambda r: (r, 0)),
        out_shape=jax.ShapeDtypeStruct((SEQ, H_DIM), jnp.float32),
    )(h, attn, wo)


_RB = 256
_NRB = SEQ // _RB


def _shared_body(a_ref, h_ref, n2s_ref, n2b_ref, up_ref, dn_ref, rt_ref,
                 y_ref, x_ref, w_ref):
    xx = _ln(a_ref[...], n2s_ref[...], n2b_ref[...])
    g = _gelu(_dot_fast(xx, up_ref[...]))
    y_ref[...] = _dot_fast(g, dn_ref[...]) + xx + h_ref[...]
    x_ref[...] = xx
    logits = _dot(xx, rt_ref[...])
    p = jnp.exp(logits - jnp.max(logits, axis=1, keepdims=True))
    p = p / jnp.sum(p, axis=1, keepdims=True)
    m1 = jnp.max(p, axis=1, keepdims=True)
    p_wo1 = jnp.where(p >= m1, -jnp.inf, p)
    m2 = jnp.max(p_wo1, axis=1, keepdims=True)
    w_ref[...] = jnp.where(p >= m2, p, 0.0)


def _shared_and_router(a, h, n2s, n2b, sh_up, sh_down, router):
    return pl.pallas_call(
        _shared_body,
        grid=(_NRB,),
        in_specs=[
            pl.BlockSpec((_RB, H_DIM), lambda r: (r, 0)),
            pl.BlockSpec((_RB, H_DIM), lambda r: (r, 0)),
            pl.BlockSpec((1, H_DIM), lambda r: (0, 0)),
            pl.BlockSpec((1, H_DIM), lambda r: (0, 0)),
            pl.BlockSpec((H_DIM, E_DIM), lambda r: (0, 0)),
            pl.BlockSpec((E_DIM, H_DIM), lambda r: (0, 0)),
            pl.BlockSpec((H_DIM, N_ROUTED), lambda r: (0, 0)),
        ],
        out_specs=[
            pl.BlockSpec((_RB, H_DIM), lambda r: (r, 0)),
            pl.BlockSpec((_RB, H_DIM), lambda r: (r, 0)),
            pl.BlockSpec((_RB, N_ROUTED), lambda r: (r, 0)),
        ],
        out_shape=[
            jax.ShapeDtypeStruct((SEQ, H_DIM), jnp.float32),
            jax.ShapeDtypeStruct((SEQ, H_DIM), jnp.float32),
            jax.ShapeDtypeStruct((SEQ, N_ROUTED), jnp.float32),
        ],
    )(a, h, n2s, n2b, sh_up, sh_down, router)


N_PAIRS = TOP_K * SEQ


def _route_meta_body(w_ref, inv1_ref, inv2_ref, wt1_ref, wt2_ref, offs_ref):
    w = w_ref[...]
    abool = w > 0.0
    ai = abool.astype(jnp.int32)
    c = ai
    for k in (1, 2, 4):
        c = c + jnp.pad(c, ((0, 0), (k, 0)))[:, :N_ROUTED]
    first = jnp.logical_and(abool, c == 1)
    second = jnp.logical_and(abool, c == 2)
    m2 = jnp.logical_or(first, second).astype(jnp.int32)
    cnt = jnp.sum(m2, axis=0, keepdims=True)
    oc = cnt
    for k in (1, 2, 4):
        oc = oc + jnp.pad(oc, ((0, 0), (k, 0)))[:, :N_ROUTED]
    offs = oc - cnt
    rk = m2
    k = 1
    while k < SEQ:
        rk = rk + jnp.pad(rk, ((k, 0), (0, 0)))[:SEQ, :]
        k *= 2
    pos = offs + (rk - m2)
    fi = first.astype(jnp.int32)
    si = second.astype(jnp.int32)
    inv1_ref[...] = jnp.sum(pos * fi, axis=1, keepdims=True)
    inv2_ref[...] = jnp.sum(pos * si, axis=1, keepdims=True)
    wt1_ref[...] = jnp.sum(w * fi.astype(jnp.float32), axis=1, keepdims=True)
    wt2_ref[...] = jnp.sum(w * si.astype(jnp.float32), axis=1, keepdims=True)
    offs_ref[...] = offs


def _route_meta(w):
    return pl.pallas_call(
        _route_meta_body,
        out_shape=[
            jax.ShapeDtypeStruct((SEQ, 1), jnp.int32),
            jax.ShapeDtypeStruct((SEQ, 1), jnp.int32),
            jax.ShapeDtypeStruct((SEQ, 1), jnp.float32),
            jax.ShapeDtypeStruct((SEQ, 1), jnp.float32),
            jax.ShapeDtypeStruct((1, N_ROUTED), jnp.int32),
        ],
    )(w)


def _sc_scatter_x(xx, idx3):
    mesh = plsc.VectorSubcoreMesh(core_axis_name="c", subcore_axis_name="s")

    @functools.partial(
        pl.kernel, mesh=mesh,
        out_type=jax.ShapeDtypeStruct((N_PAIRS, H_DIM), jnp.float32),
        scratch_types=[
            pltpu.VMEM((TOP_K, _B_PER_W), jnp.int32),
            pltpu.VMEM((_B_PER_W, H_DIM), jnp.float32),
        ],
    )
    def k(xx_hbm, idx_hbm, out_hbm, idx_v, rows_v):
        wid = lax.axis_index("s") * 2 + lax.axis_index("c")
        base = wid * _B_PER_W
        pltpu.sync_copy(idx_hbm.at[wid], idx_v)
        pltpu.sync_copy(xx_hbm.at[pl.ds(base, _B_PER_W)], rows_v)
        pltpu.sync_copy(rows_v, out_hbm.at[idx_v.at[0]])
        pltpu.sync_copy(rows_v, out_hbm.at[idx_v.at[1]])

    return k(xx, idx3)


def _sc_gather2(os_, idx3):
    mesh = plsc.VectorSubcoreMesh(core_axis_name="c", subcore_axis_name="s")

    @functools.partial(
        pl.kernel, mesh=mesh,
        out_type=[
            jax.ShapeDtypeStruct((SEQ, H_DIM), jnp.float32),
            jax.ShapeDtypeStruct((SEQ, H_DIM), jnp.float32),
        ],
        scratch_types=[
            pltpu.VMEM((TOP_K, _B_PER_W), jnp.int32),
            pltpu.VMEM((_B_PER_W, H_DIM), jnp.float32),
        ],
    )
    def k(os_hbm, idx_hbm, g1_hbm, g2_hbm, idx_v, rows_v):
        wid = lax.axis_index("s") * 2 + lax.axis_index("c")
        base = wid * _B_PER_W
        pltpu.sync_copy(idx_hbm.at[wid], idx_v)
        pltpu.sync_copy(os_hbm.at[idx_v.at[0]], rows_v)
        pltpu.sync_copy(rows_v, g1_hbm.at[pl.ds(base, _B_PER_W)])
        pltpu.sync_copy(os_hbm.at[idx_v.at[1]], rows_v)
        pltpu.sync_copy(rows_v, g2_hbm.at[pl.ds(base, _B_PER_W)])

    return k(os_, idx3)


_TB = 256
_NT = N_PAIRS // _TB
_NWORK = _NT + N_ROUTED - 1


def _etile_body(meta_ref, x_ref, up_ref, dn_ref, out_ref):
    wi = pl.program_id(0)
    tile = meta_ref[0, wi]
    start = meta_ref[2, wi]
    end = meta_ref[3, wi]
    fst = meta_ref[4, wi]
    rows = tile * _TB + lax.broadcasted_iota(jnp.int32, (_TB, 1), 0)
    m = jnp.logical_and(rows >= start, rows < end)
    xm = jnp.where(m, x_ref[...], 0.0)
    t = _dot_fast(_gelu(_dot_fast(xm, up_ref[0])), dn_ref[0])

    @pl.when(fst == 1)
    def _():
        out_ref[...] = t

    @pl.when(fst == 0)
    def _():
        out_ref[...] = out_ref[...] + t


def _expert_tiles(meta, xs, r_up, r_down):
    grid_spec = pltpu.PrefetchScalarGridSpec(
        num_scalar_prefetch=1,
        grid=(_NWORK,),
        in_specs=[
            pl.BlockSpec((_TB, H_DIM), lambda wi, m: (m[0, wi], 0)),
            pl.BlockSpec((1, H_DIM, E_DIM), lambda wi, m: (m[1, wi], 0, 0)),
            pl.BlockSpec((1, E_DIM, H_DIM), lambda wi, m: (m[1, wi], 0, 0)),
        ],
        out_specs=pl.BlockSpec((_TB, H_DIM), lambda wi, m: (m[0, wi], 0)),
    )
    return pl.pallas_call(
        _etile_body,
        grid_spec=grid_spec,
        out_shape=jax.ShapeDtypeStruct((N_PAIRS, H_DIM), jnp.float32),
    )(meta, xs, r_up, r_down)


def _combine_body(y_ref, g1_ref, g2_ref, wt1_ref, wt2_ref, nfs_ref, nfb_ref,
                  out_ref):
    routed = g1_ref[...] * wt1_ref[...] + g2_ref[...] * wt2_ref[...]
    out_ref[...] = _ln(routed + y_ref[...], nfs_ref[...], nfb_ref[...])


def _combine_final_ln(y, g1, g2, wt1, wt2, nfs, nfb):
    return pl.pallas_call(
        _combine_body,
        grid=(_NRB,),
        in_specs=[
            pl.BlockSpec((_RB, H_DIM), lambda r: (r, 0)),
            pl.BlockSpec((_RB, H_DIM), lambda r: (r, 0)),
            pl.BlockSpec((_RB, H_DIM), lambda r: (r, 0)),
            pl.BlockSpec((_RB, 1), lambda r: (r, 0)),
            pl.BlockSpec((_RB, 1), lambda r: (r, 0)),
            pl.BlockSpec((1, H_DIM), lambda r: (0, 0)),
            pl.BlockSpec((1, H_DIM), lambda r: (0, 0)),
        ],
        out_specs=pl.BlockSpec((_RB, H_DIM), lambda r: (r, 0)),
        out_shape=jax.ShapeDtypeStruct((SEQ, H_DIM), jnp.float32),
    )(y, g1, g2, wt1, wt2, nfs, nfb)


_CB = 1280
_NCB = N_TOKENS // _CB


def _head_body(mf_ref, proj_ref, out_ref):
    out_ref[...] = _dot_fast(mf_ref[...], proj_ref[...])


_HRB = 512
_NHRB = SEQ // _HRB


def _lm_head(mf, proj):
    return pl.pallas_call(
        _head_body,
        grid=(_NCB, _NHRB),
        in_specs=[
            pl.BlockSpec((_HRB, H_DIM), lambda c, r: (r, 0)),
            pl.BlockSpec((H_DIM, _CB), lambda c, r: (0, c)),
        ],
        out_specs=pl.BlockSpec((_HRB, _CB), lambda c, r: (r, c)),
        out_shape=jax.ShapeDtypeStruct((SEQ, N_TOKENS), jnp.float32),
    )(mf, proj)


def kernel(x, emb, n1_scale, n1_bias, W_dkv, W_dq, W_uk, W_uv, W_uq, W_kr,
           W_qr, W_o, n2_scale, n2_bias, sh_up, sh_down, r_up, r_down,
           router, nf_scale, nf_bias, proj):
    idx = x.reshape(SEQ)
    h = _emb_gather(emb, idx)
    n1s = n1_scale.reshape(1, H_DIM)
    n1b = n1_bias.reshape(1, H_DIM)
    a = _attn_block(h, n1s, n1b, W_dkv, W_dq, W_uk, W_uv, W_uq, W_kr, W_qr,
                    W_o)
    y, xx, w = _shared_and_router(a, h, n2_scale.reshape(1, H_DIM),
                                  n2_bias.reshape(1, H_DIM), sh_up, sh_down,
                                  router)
    inv1, inv2, wt1, wt2, offs = _route_meta(w)
    offs9 = jnp.concatenate([offs.reshape(N_ROUTED),
                             jnp.array([N_PAIRS], jnp.int32)])
    tau0 = jnp.arange(_NT, dtype=jnp.int32) * _TB
    flag = jnp.logical_and(offs9[None, :N_ROUTED] < (tau0 + _TB)[:, None],
                           offs9[None, 1:] > tau0[:, None])
    flat = flag.reshape(_NT * N_ROUTED)
    order = jnp.argsort(jnp.logical_not(flat), stable=True).astype(jnp.int32)
    sel = order[:_NWORK]
    nact = jnp.sum(flat.astype(jnp.int32))
    valid = jnp.arange(_NWORK, dtype=jnp.int32) < nact
    tile_w = jnp.where(valid, sel // N_ROUTED, _NT - 1)
    e_w = jnp.where(valid, sel % N_ROUTED, N_ROUTED - 1)
    s_w = jnp.where(valid, jnp.take(offs9, e_w), 0)
    en_w = jnp.where(valid, jnp.take(offs9, e_w + 1), 0)
    fst_w = jnp.where(
        valid,
        jnp.concatenate([jnp.array([1], jnp.int32),
                         (tile_w[1:] != tile_w[:-1]).astype(jnp.int32)]),
        0)
    meta = jnp.stack([tile_w, e_w, s_w, en_w, fst_w]).astype(jnp.int32)
    idx3 = jnp.stack([inv1.reshape(_NW, _B_PER_W),
                      inv2.reshape(_NW, _B_PER_W)], axis=1)
    xs = _sc_scatter_x(xx, idx3)
    os_ = _expert_tiles(meta, xs, r_up, r_down)
    g1, g2 = _sc_gather2(os_, idx3)
    mf = _combine_final_ln(y, g1, g2, wt1, wt2, nf_scale.reshape(1, H_DIM),
                           nf_bias.reshape(1, H_DIM))
    logits = _lm_head(mf, proj)
    return logits[None]


# speedup vs baseline: 1.0931x; 1.0931x over previous
import functools
import math

import jax
import jax.numpy as jnp
import numpy as np
from jax import lax
from jax.experimental import pallas as pl
from jax.experimental.pallas import tpu as pltpu
from jax.experimental.pallas import tpu_sc as plsc

H_DIM = 768
N_HEADS = 12
C_DIM = 256
N_TOKENS = 32000
N_ROUTED = 8
TOP_K = 2
E_DIM = 4 * H_DIM
D_HEAD = H_DIM // N_HEADS
D_ROPE = D_HEAD // 2
UP_DIM = (D_HEAD - D_ROPE) * N_HEADS
SEQ = 2048

PREC = jax.lax.Precision.DEFAULT

_half = D_ROPE // 2
_inv_freq = 1.0 / (10000.0 ** (np.arange(0, _half, 2, dtype=np.float32) / _half))
_t = np.arange(SEQ, dtype=np.float32) / 40.0
_freqs = _t[:, None] * _inv_freq[None, :]
_c8 = np.cos(_freqs)
_s8 = np.sin(_freqs)
_ones16 = np.ones((SEQ, _half), np.float32)
_zeros16 = np.zeros((SEQ, _half), np.float32)
_COSF = np.tile(np.concatenate([_c8, _c8, _ones16], axis=1), (1, N_HEADS))
_SINF = np.tile(np.concatenate([_s8, _s8, _zeros16], axis=1), (1, N_HEADS))
_P = np.zeros((UP_DIM, UP_DIM), np.float32)
for _h in range(N_HEADS):
    _b = _h * D_ROPE
    for _i in range(8):
        _P[_b + 8 + _i, _b + _i] = -1.0
        _P[_b + _i, _b + 8 + _i] = 1.0
COSF = _COSF
SINF = _SINF
PMAT = _P


def _dot(a, b):
    return lax.dot_general(a, b, (((1,), (0,)), ((), ())),
                           preferred_element_type=jnp.float32, precision=PREC)


def _dot_fast(a, b):
    return lax.dot_general(a.astype(jnp.bfloat16), b.astype(jnp.bfloat16),
                           (((1,), (0,)), ((), ())),
                           preferred_element_type=jnp.float32)


def _dot_x(a, b):
    return lax.dot_general(a, b, (((1,), (0,)), ((), ())),
                           preferred_element_type=jnp.float32,
                           precision=jax.lax.Precision.HIGHEST)


def _ln(x, scale, bias):
    m = jnp.mean(x, axis=-1, keepdims=True)
    d = x - m
    v = jnp.mean(d * d, axis=-1, keepdims=True)
    return d * lax.rsqrt(v + 1e-5) * scale + bias


def _erf(x):
    ax = jnp.abs(x)
    t = 1.0 / (1.0 + 0.3275911 * ax)
    y = ((((1.061405429 * t - 1.453152027) * t + 1.421413741) * t
          - 0.284496736) * t + 0.254829592) * t
    y = 1.0 - y * jnp.exp(-ax * ax)
    return jnp.sign(x) * y


def _gelu(x):
    return 0.5 * x * (1.0 + _erf(x * (1.0 / math.sqrt(2.0))))


_NW = 32
_B_PER_W = SEQ // _NW


def _emb_gather(emb, idx):
    mesh = plsc.VectorSubcoreMesh(core_axis_name="c", subcore_axis_name="s")

    @functools.partial(
        pl.kernel, mesh=mesh,
        out_type=jax.ShapeDtypeStruct((SEQ, H_DIM), jnp.float32),
        scratch_types=[
            pltpu.VMEM((_B_PER_W,), jnp.int32),
            pltpu.VMEM((_B_PER_W, H_DIM), jnp.float32),
            pltpu.SemaphoreType.DMA,
        ],
    )
    def k(emb_hbm, idx_hbm, out_hbm, idx_v, rows_v, sem):
        wid = lax.axis_index("s") * 2 + lax.axis_index("c")
        base = wid * _B_PER_W
        pltpu.sync_copy(idx_hbm.at[pl.ds(base, _B_PER_W)], idx_v)
        pltpu.async_copy(emb_hbm.at[idx_v], rows_v, sem).wait()
        pltpu.sync_copy(rows_v, out_hbm.at[pl.ds(base, _B_PER_W)])

    return k(emb, idx)


_AB = 256
_NAB = SEQ // _AB


def _attn1_body(h_ref, n1s_ref, n1b_ref, wdkv_ref, wdq_ref, wuk_ref,
                wuv_ref, wuq_ref, wkr_ref, wqr_ref, cos_ref, sin_ref,
                p_ref, kc_ref, kr_ref, qc_ref, qr_ref, v_ref):
    h1 = _ln(h_ref[...], n1s_ref[...], n1b_ref[...])
    c_kv = _dot(h1, wdkv_ref[...])
    c_q = _dot(h1, wdq_ref[...])
    qc_ref[...] = _dot(c_q, wuq_ref[...])
    kc_ref[...] = _dot(c_kv, wuk_ref[...])
    v_ref[...] = _dot(c_kv, wuv_ref[...])
    k_r = _dot(h1, wkr_ref[...])
    q_r = _dot(c_q, wqr_ref[...])
    cos = cos_ref[...]
    sin = sin_ref[...]
    pm = p_ref[...]
    kr_ref[...] = k_r * cos + _dot_x(k_r, pm) * sin
    qr_ref[...] = q_r * cos + _dot_x(q_r, pm) * sin


def _attn2_body(kc_ref, kr_ref, qc_ref, qr_ref, v_ref, out_ref):
    kc = kc_ref[...]
    kr = kr_ref[...]
    qc = qc_ref[...]
    qr = qr_ref[...]
    v = v_ref[...]
    outs = []
    for hh in range(N_HEADS):
        s32 = slice(hh * D_ROPE, (hh + 1) * D_ROPE)
        s64 = slice(hh * D_HEAD, (hh + 1) * D_HEAD)
        k_cat = jnp.concatenate([kc[:, s32], kr[:, s32]], axis=1)
        q_cat = jnp.concatenate([qc[:, s32], qr[:, s32]], axis=1)
        s = lax.dot_general(k_cat, q_cat, (((1,), (1,)), ((), ())),
                            preferred_element_type=jnp.float32,
                            precision=PREC) * (1.0 / math.sqrt(D_HEAD))
        outs.append(_dot(s, v[:, s64]))
    out_ref[...] = jnp.concatenate(outs, axis=1)


def _attn3_body(h_ref, attn_ref, wo_ref, out_ref):
    out_ref[...] = _dot(attn_ref[...], wo_ref[...]) + h_ref[...]


def _attn_block(h, n1s, n1b, wdkv, wdq, wuk, wuv, wuq, wkr, wqr, wo):
    kc, kr, qc, qr, v = pl.pallas_call(
        _attn1_body,
        grid=(_NAB,),
        in_specs=[
            pl.BlockSpec((_AB, H_DIM), lambda r: (r, 0)),
            pl.BlockSpec((1, H_DIM), lambda r: (0, 0)),
            pl.BlockSpec((1, H_DIM), lambda r: (0, 0)),
            pl.BlockSpec((H_DIM, C_DIM), lambda r: (0, 0)),
            pl.BlockSpec((H_DIM, C_DIM), lambda r: (0, 0)),
            pl.BlockSpec((C_DIM, UP_DIM), lambda r: (0, 0)),
            pl.BlockSpec((C_DIM, H_DIM), lambda r: (0, 0)),
            pl.BlockSpec((C_DIM, UP_DIM), lambda r: (0, 0)),
            pl.BlockSpec((H_DIM, UP_DIM), lambda r: (0, 0)),
            pl.BlockSpec((C_DIM, UP_DIM), lambda r: (0, 0)),
            pl.BlockSpec((_AB, UP_DIM), lambda r: (r, 0)),
            pl.BlockSpec((_AB, UP_DIM), lambda r: (r, 0)),
            pl.BlockSpec((UP_DIM, UP_DIM), lambda r: (0, 0)),
        ],
        out_specs=[
            pl.BlockSpec((_AB, UP_DIM), lambda r: (r, 0)),
            pl.BlockSpec((_AB, UP_DIM), lambda r: (r, 0)),
            pl.BlockSpec((_AB, UP_DIM), lambda r: (r, 0)),
            pl.BlockSpec((_AB, UP_DIM), lambda r: (r, 0)),
            pl.BlockSpec((_AB, H_DIM), lambda r: (r, 0)),
        ],
        out_shape=[
            jax.ShapeDtypeStruct((SEQ, UP_DIM), jnp.float32),
            jax.ShapeDtypeStruct((SEQ, UP_DIM), jnp.float32),
            jax.ShapeDtypeStruct((SEQ, UP_DIM), jnp.float32),
            jax.ShapeDtypeStruct((SEQ, UP_DIM), jnp.float32),
            jax.ShapeDtypeStruct((SEQ, H_DIM), jnp.float32),
        ],
    )(h, n1s, n1b, wdkv, wdq, wuk, wuv, wuq, wkr, wqr, COSF, SINF, PMAT)
    attn = pl.pallas_call(
        _attn2_body,
        grid=(_NAB,),
        in_specs=[
            pl.BlockSpec((_AB, UP_DIM), lambda i: (i, 0)),
            pl.BlockSpec((_AB, UP_DIM), lambda i: (i, 0)),
            pl.BlockSpec((SEQ, UP_DIM), lambda i: (0, 0)),
            pl.BlockSpec((SEQ, UP_DIM), lambda i: (0, 0)),
            pl.BlockSpec((SEQ, H_DIM), lambda i: (0, 0)),
        ],
        out_specs=pl.BlockSpec((_AB, H_DIM), lambda i: (i, 0)),
        out_shape=jax.ShapeDtypeStruct((SEQ, H_DIM), jnp.float32),
    )(kc, kr, qc, qr, v)
    return pl.pallas_call(
        _attn3_body,
        grid=(_NAB,),
        in_specs=[
            pl.BlockSpec((_AB, H_DIM), lambda r: (r, 0)),
            pl.BlockSpec((_AB, H_DIM), lambda r: (r, 0)),
            pl.BlockSpec((H_DIM, H_DIM), lambda r: (0, 0)),
        ],
        out_specs=pl.BlockSpec((_AB, H_DIM), lambda r: (r, 0)),
        out_shape=jax.ShapeDtypeStruct((SEQ, H_DIM), jnp.float32),
    )(h, attn, wo)


_RB = 256
_NRB = SEQ // _RB


def _shared_body(a_ref, h_ref, n2s_ref, n2b_ref, up_ref, dn_ref, rt_ref,
                 y_ref, x_ref, w_ref):
    xx = _ln(a_ref[...], n2s_ref[...], n2b_ref[...])
    g = _gelu(_dot(xx, up_ref[...]))
    y_ref[...] = _dot(g, dn_ref[...]) + xx + h_ref[...]
    x_ref[...] = xx
    logits = _dot(xx, rt_ref[...])
    p = jnp.exp(logits - jnp.max(logits, axis=1, keepdims=True))
    p = p / jnp.sum(p, axis=1, keepdims=True)
    m1 = jnp.max(p, axis=1, keepdims=True)
    p_wo1 = jnp.where(p >= m1, -jnp.inf, p)
    m2 = jnp.max(p_wo1, axis=1, keepdims=True)
    w_ref[...] = jnp.where(p >= m2, p, 0.0)


def _shared_and_router(a, h, n2s, n2b, sh_up, sh_down, router):
    return pl.pallas_call(
        _shared_body,
        grid=(_NRB,),
        in_specs=[
            pl.BlockSpec((_RB, H_DIM), lambda r: (r, 0)),
            pl.BlockSpec((_RB, H_DIM), lambda r: (r, 0)),
            pl.BlockSpec((1, H_DIM), lambda r: (0, 0)),
            pl.BlockSpec((1, H_DIM), lambda r: (0, 0)),
            pl.BlockSpec((H_DIM, E_DIM), lambda r: (0, 0)),
            pl.BlockSpec((E_DIM, H_DIM), lambda r: (0, 0)),
            pl.BlockSpec((H_DIM, N_ROUTED), lambda r: (0, 0)),
        ],
        out_specs=[
            pl.BlockSpec((_RB, H_DIM), lambda r: (r, 0)),
            pl.BlockSpec((_RB, H_DIM), lambda r: (r, 0)),
            pl.BlockSpec((_RB, N_ROUTED), lambda r: (r, 0)),
        ],
        out_shape=[
            jax.ShapeDtypeStruct((SEQ, H_DIM), jnp.float32),
            jax.ShapeDtypeStruct((SEQ, H_DIM), jnp.float32),
            jax.ShapeDtypeStruct((SEQ, N_ROUTED), jnp.float32),
        ],
    )(a, h, n2s, n2b, sh_up, sh_down, router)


N_PAIRS = TOP_K * SEQ


def _route_meta_body(w_ref, inv1_ref, inv2_ref, wt1_ref, wt2_ref, offs_ref):
    w = w_ref[...]
    abool = w > 0.0
    ai = abool.astype(jnp.int32)
    c = ai
    for k in (1, 2, 4):
        c = c + jnp.pad(c, ((0, 0), (k, 0)))[:, :N_ROUTED]
    first = jnp.logical_and(abool, c == 1)
    second = jnp.logical_and(abool, c == 2)
    m2 = jnp.logical_or(first, second).astype(jnp.int32)
    cnt = jnp.sum(m2, axis=0, keepdims=True)
    oc = cnt
    for k in (1, 2, 4):
        oc = oc + jnp.pad(oc, ((0, 0), (k, 0)))[:, :N_ROUTED]
    offs = oc - cnt
    rk = m2
    k = 1
    while k < SEQ:
        rk = rk + jnp.pad(rk, ((k, 0), (0, 0)))[:SEQ, :]
        k *= 2
    pos = offs + (rk - m2)
    fi = first.astype(jnp.int32)
    si = second.astype(jnp.int32)
    inv1_ref[...] = jnp.sum(pos * fi, axis=1, keepdims=True)
    inv2_ref[...] = jnp.sum(pos * si, axis=1, keepdims=True)
    wt1_ref[...] = jnp.sum(w * fi.astype(jnp.float32), axis=1, keepdims=True)
    wt2_ref[...] = jnp.sum(w * si.astype(jnp.float32), axis=1, keepdims=True)
    offs_ref[...] = offs


def _route_meta(w):
    return pl.pallas_call(
        _route_meta_body,
        out_shape=[
            jax.ShapeDtypeStruct((SEQ, 1), jnp.int32),
            jax.ShapeDtypeStruct((SEQ, 1), jnp.int32),
            jax.ShapeDtypeStruct((SEQ, 1), jnp.float32),
            jax.ShapeDtypeStruct((SEQ, 1), jnp.float32),
            jax.ShapeDtypeStruct((1, N_ROUTED), jnp.int32),
        ],
    )(w)


def _sc_scatter_x(xx, idx3):
    mesh = plsc.VectorSubcoreMesh(core_axis_name="c", subcore_axis_name="s")

    @functools.partial(
        pl.kernel, mesh=mesh,
        out_type=jax.ShapeDtypeStruct((N_PAIRS, H_DIM), jnp.float32),
        scratch_types=[
            pltpu.VMEM((TOP_K, _B_PER_W), jnp.int32),
            pltpu.VMEM((_B_PER_W, H_DIM), jnp.float32),
        ],
    )
    def k(xx_hbm, idx_hbm, out_hbm, idx_v, rows_v):
        wid = lax.axis_index("s") * 2 + lax.axis_index("c")
        base = wid * _B_PER_W
        pltpu.sync_copy(idx_hbm.at[wid], idx_v)
        pltpu.sync_copy(xx_hbm.at[pl.ds(base, _B_PER_W)], rows_v)
        pltpu.sync_copy(rows_v, out_hbm.at[idx_v.at[0]])
        pltpu.sync_copy(rows_v, out_hbm.at[idx_v.at[1]])

    return k(xx, idx3)


def _sc_gather2(os_, idx3):
    mesh = plsc.VectorSubcoreMesh(core_axis_name="c", subcore_axis_name="s")

    @functools.partial(
        pl.kernel, mesh=mesh,
        out_type=[
            jax.ShapeDtypeStruct((SEQ, H_DIM), jnp.float32),
            jax.ShapeDtypeStruct((SEQ, H_DIM), jnp.float32),
        ],
        scratch_types=[
            pltpu.VMEM((TOP_K, _B_PER_W), jnp.int32),
            pltpu.VMEM((_B_PER_W, H_DIM), jnp.float32),
        ],
    )
    def k(os_hbm, idx_hbm, g1_hbm, g2_hbm, idx_v, rows_v):
        wid = lax.axis_index("s") * 2 + lax.axis_index("c")
        base = wid * _B_PER_W
        pltpu.sync_copy(idx_hbm.at[wid], idx_v)
        pltpu.sync_copy(os_hbm.at[idx_v.at[0]], rows_v)
        pltpu.sync_copy(rows_v, g1_hbm.at[pl.ds(base, _B_PER_W)])
        pltpu.sync_copy(os_hbm.at[idx_v.at[1]], rows_v)
        pltpu.sync_copy(rows_v, g2_hbm.at[pl.ds(base, _B_PER_W)])

    return k(os_, idx3)


_TB = 128
_NT = N_PAIRS // _TB
_NWORK = _NT + N_ROUTED - 1


def _etile_body(meta_ref, x_ref, up_ref, dn_ref, out_ref):
    wi = pl.program_id(0)
    tile = meta_ref[0, wi]
    start = meta_ref[2, wi]
    end = meta_ref[3, wi]
    fst = meta_ref[4, wi]
    rows = tile * _TB + lax.broadcasted_iota(jnp.int32, (_TB, 1), 0)
    m = jnp.logical_and(rows >= start, rows < end)
    xm = jnp.where(m, x_ref[...], 0.0)
    t = _dot(_gelu(_dot(xm, up_ref[0])), dn_ref[0])

    @pl.when(fst == 1)
    def _():
        out_ref[...] = t

    @pl.when(fst == 0)
    def _():
        out_ref[...] = out_ref[...] + t


def _expert_tiles(meta, xs, r_up, r_down):
    grid_spec = pltpu.PrefetchScalarGridSpec(
        num_scalar_prefetch=1,
        grid=(_NWORK,),
        in_specs=[
            pl.BlockSpec((_TB, H_DIM), lambda wi, m: (m[0, wi], 0)),
            pl.BlockSpec((1, H_DIM, E_DIM), lambda wi, m: (m[1, wi], 0, 0)),
            pl.BlockSpec((1, E_DIM, H_DIM), lambda wi, m: (m[1, wi], 0, 0)),
        ],
        out_specs=pl.BlockSpec((_TB, H_DIM), lambda wi, m: (m[0, wi], 0)),
    )
    return pl.pallas_call(
        _etile_body,
        grid_spec=grid_spec,
        out_shape=jax.ShapeDtypeStruct((N_PAIRS, H_DIM), jnp.float32),
    )(meta, xs, r_up, r_down)


def _combine_body(y_ref, g1_ref, g2_ref, wt1_ref, wt2_ref, nfs_ref, nfb_ref,
                  out_ref):
    routed = g1_ref[...] * wt1_ref[...] + g2_ref[...] * wt2_ref[...]
    out_ref[...] = _ln(routed + y_ref[...], nfs_ref[...], nfb_ref[...])


def _combine_final_ln(y, g1, g2, wt1, wt2, nfs, nfb):
    return pl.pallas_call(
        _combine_body,
        grid=(_NRB,),
        in_specs=[
            pl.BlockSpec((_RB, H_DIM), lambda r: (r, 0)),
            pl.BlockSpec((_RB, H_DIM), lambda r: (r, 0)),
            pl.BlockSpec((_RB, H_DIM), lambda r: (r, 0)),
            pl.BlockSpec((_RB, 1), lambda r: (r, 0)),
            pl.BlockSpec((_RB, 1), lambda r: (r, 0)),
            pl.BlockSpec((1, H_DIM), lambda r: (0, 0)),
            pl.BlockSpec((1, H_DIM), lambda r: (0, 0)),
        ],
        out_specs=pl.BlockSpec((_RB, H_DIM), lambda r: (r, 0)),
        out_shape=jax.ShapeDtypeStruct((SEQ, H_DIM), jnp.float32),
    )(y, g1, g2, wt1, wt2, nfs, nfb)


_CB = 3200
_NCB = N_TOKENS // _CB


def _head_body(mf_ref, proj_ref, out_ref):
    out_ref[...] = _dot(mf_ref[...], proj_ref[...])


_HRB = 512
_NHRB = SEQ // _HRB


def _lm_head(mf, proj):
    return pl.pallas_call(
        _head_body,
        grid=(_NCB, _NHRB),
        in_specs=[
            pl.BlockSpec((_HRB, H_DIM), lambda c, r: (r, 0)),
            pl.BlockSpec((H_DIM, _CB), lambda c, r: (0, c)),
        ],
        out_specs=pl.BlockSpec((_HRB, _CB), lambda c, r: (r, c)),
        out_shape=jax.ShapeDtypeStruct((SEQ, N_TOKENS), jnp.float32),
    )(mf, proj)


def kernel(x, emb, n1_scale, n1_bias, W_dkv, W_dq, W_uk, W_uv, W_uq, W_kr,
           W_qr, W_o, n2_scale, n2_bias, sh_up, sh_down, r_up, r_down,
           router, nf_scale, nf_bias, proj):
    idx = x.reshape(SEQ)
    h = _emb_gather(emb, idx)
    n1s = n1_scale.reshape(1, H_DIM)
    n1b = n1_bias.reshape(1, H_DIM)
    a = _attn_block(h, n1s, n1b, W_dkv, W_dq, W_uk, W_uv, W_uq, W_kr, W_qr,
                    W_o)
    y, xx, w = _shared_and_router(a, h, n2_scale.reshape(1, H_DIM),
                                  n2_bias.reshape(1, H_DIM), sh_up, sh_down,
                                  router)
    inv1, inv2, wt1, wt2, offs = _route_meta(w)
    offs9 = jnp.concatenate([offs.reshape(N_ROUTED),
                             jnp.array([N_PAIRS], jnp.int32)])
    tau0 = jnp.arange(_NT, dtype=jnp.int32) * _TB
    flag = jnp.logical_and(offs9[None, :N_ROUTED] < (tau0 + _TB)[:, None],
                           offs9[None, 1:] > tau0[:, None])
    flat = flag.reshape(_NT * N_ROUTED)
    order = jnp.argsort(jnp.logical_not(flat), stable=True).astype(jnp.int32)
    sel = order[:_NWORK]
    nact = jnp.sum(flat.astype(jnp.int32))
    valid = jnp.arange(_NWORK, dtype=jnp.int32) < nact
    tile_w = jnp.where(valid, sel // N_ROUTED, _NT - 1)
    e_w = jnp.where(valid, sel % N_ROUTED, N_ROUTED - 1)
    s_w = jnp.where(valid, jnp.take(offs9, e_w), 0)
    en_w = jnp.where(valid, jnp.take(offs9, e_w + 1), 0)
    fst_w = jnp.where(
        valid,
        jnp.concatenate([jnp.array([1], jnp.int32),
                         (tile_w[1:] != tile_w[:-1]).astype(jnp.int32)]),
        0)
    meta = jnp.stack([tile_w, e_w, s_w, en_w, fst_w]).astype(jnp.int32)
    idx3 = jnp.stack([inv1.reshape(_NW, _B_PER_W),
                      inv2.reshape(_NW, _B_PER_W)], axis=1)
    xs = _sc_scatter_x(xx, idx3)
    os_ = _expert_tiles(meta, xs, r_up, r_down)
    g1, g2 = _sc_gather2(os_, idx3)
    mf = _combine_final_ln(y, g1, g2, wt1, wt2, nf_scale.reshape(1, H_DIM),
                           nf_bias.reshape(1, H_DIM))
    logits = _lm_head(mf, proj)
    return logits[None]


# speedup vs baseline: 1.1363x; 1.0396x over previous
import functools
import math

import jax
import jax.numpy as jnp
import numpy as np
from jax import lax
from jax.experimental import pallas as pl
from jax.experimental.pallas import tpu as pltpu
from jax.experimental.pallas import tpu_sc as plsc

H_DIM = 768
N_HEADS = 12
C_DIM = 256
N_TOKENS = 32000
N_ROUTED = 8
TOP_K = 2
E_DIM = 4 * H_DIM
D_HEAD = H_DIM // N_HEADS
D_ROPE = D_HEAD // 2
UP_DIM = (D_HEAD - D_ROPE) * N_HEADS
SEQ = 2048

PREC = jax.lax.Precision.DEFAULT

_half = D_ROPE // 2
_inv_freq = 1.0 / (10000.0 ** (np.arange(0, _half, 2, dtype=np.float32) / _half))
_t = np.arange(SEQ, dtype=np.float32) / 40.0
_freqs = _t[:, None] * _inv_freq[None, :]
_c8 = np.cos(_freqs)
_s8 = np.sin(_freqs)
_ones16 = np.ones((SEQ, _half), np.float32)
_zeros16 = np.zeros((SEQ, _half), np.float32)
_COSF = np.tile(np.concatenate([_c8, _c8, _ones16], axis=1), (1, N_HEADS))
_SINF = np.tile(np.concatenate([_s8, _s8, _zeros16], axis=1), (1, N_HEADS))
_P = np.zeros((UP_DIM, UP_DIM), np.float32)
for _h in range(N_HEADS):
    _b = _h * D_ROPE
    for _i in range(8):
        _P[_b + 8 + _i, _b + _i] = -1.0
        _P[_b + _i, _b + 8 + _i] = 1.0
COSF = _COSF
SINF = _SINF
PMAT = _P


def _dot(a, b):
    return lax.dot_general(a, b, (((1,), (0,)), ((), ())),
                           preferred_element_type=jnp.float32, precision=PREC)


def _dot_fast(a, b):
    return lax.dot_general(a.astype(jnp.bfloat16), b.astype(jnp.bfloat16),
                           (((1,), (0,)), ((), ())),
                           preferred_element_type=jnp.float32)


def _dot_x(a, b):
    return lax.dot_general(a, b, (((1,), (0,)), ((), ())),
                           preferred_element_type=jnp.float32,
                           precision=jax.lax.Precision.HIGHEST)


def _ln(x, scale, bias):
    m = jnp.mean(x, axis=-1, keepdims=True)
    d = x - m
    v = jnp.mean(d * d, axis=-1, keepdims=True)
    return d * lax.rsqrt(v + 1e-5) * scale + bias


def _erf(x):
    ax = jnp.abs(x)
    t = 1.0 / (1.0 + 0.3275911 * ax)
    y = ((((1.061405429 * t - 1.453152027) * t + 1.421413741) * t
          - 0.284496736) * t + 0.254829592) * t
    y = 1.0 - y * jnp.exp(-ax * ax)
    return jnp.sign(x) * y


def _gelu(x):
    return 0.5 * x * (1.0 + _erf(x * (1.0 / math.sqrt(2.0))))


_NW = 32
_B_PER_W = SEQ // _NW


def _emb_gather(emb, idx):
    mesh = plsc.VectorSubcoreMesh(core_axis_name="c", subcore_axis_name="s")

    @functools.partial(
        pl.kernel, mesh=mesh,
        out_type=jax.ShapeDtypeStruct((SEQ, H_DIM), jnp.float32),
        scratch_types=[
            pltpu.VMEM((_B_PER_W,), jnp.int32),
            pltpu.VMEM((_B_PER_W, H_DIM), jnp.float32),
            pltpu.SemaphoreType.DMA,
        ],
    )
    def k(emb_hbm, idx_hbm, out_hbm, idx_v, rows_v, sem):
        wid = lax.axis_index("s") * 2 + lax.axis_index("c")
        base = wid * _B_PER_W
        pltpu.sync_copy(idx_hbm.at[pl.ds(base, _B_PER_W)], idx_v)
        pltpu.async_copy(emb_hbm.at[idx_v], rows_v, sem).wait()
        pltpu.sync_copy(rows_v, out_hbm.at[pl.ds(base, _B_PER_W)])

    return k(emb, idx)


_AB = 256
_NAB = SEQ // _AB


def _attn1_body(h_ref, n1s_ref, n1b_ref, wdkv_ref, wdq_ref, wuk_ref,
                wuv_ref, wuq_ref, wkr_ref, wqr_ref, cos_ref, sin_ref,
                p_ref, kc_ref, kr_ref, qc_ref, qr_ref, v_ref):
    h1 = _ln(h_ref[...], n1s_ref[...], n1b_ref[...])
    c_kv = _dot(h1, wdkv_ref[...])
    c_q = _dot(h1, wdq_ref[...])
    qc_ref[...] = _dot(c_q, wuq_ref[...])
    kc_ref[...] = _dot(c_kv, wuk_ref[...])
    v_ref[...] = _dot(c_kv, wuv_ref[...])
    k_r = _dot(h1, wkr_ref[...])
    q_r = _dot(c_q, wqr_ref[...])
    cos = cos_ref[...]
    sin = sin_ref[...]
    pm = p_ref[...]
    kr_ref[...] = k_r * cos + _dot_x(k_r, pm) * sin
    qr_ref[...] = q_r * cos + _dot_x(q_r, pm) * sin


def _attn2_body(kc_ref, kr_ref, qc_ref, qr_ref, v_ref, out_ref):
    kc = kc_ref[...]
    kr = kr_ref[...]
    qc = qc_ref[...]
    qr = qr_ref[...]
    v = v_ref[...]
    outs = []
    for hh in range(N_HEADS):
        s32 = slice(hh * D_ROPE, (hh + 1) * D_ROPE)
        s64 = slice(hh * D_HEAD, (hh + 1) * D_HEAD)
        k_cat = jnp.concatenate([kc[:, s32], kr[:, s32]], axis=1)
        q_cat = jnp.concatenate([qc[:, s32], qr[:, s32]], axis=1)
        s = lax.dot_general(k_cat, q_cat, (((1,), (1,)), ((), ())),
                            preferred_element_type=jnp.float32,
                            precision=PREC) * (1.0 / math.sqrt(D_HEAD))
        outs.append(_dot(s, v[:, s64]))
    out_ref[...] = jnp.concatenate(outs, axis=1)


def _attn3_body(h_ref, attn_ref, wo_ref, out_ref):
    out_ref[...] = _dot(attn_ref[...], wo_ref[...]) + h_ref[...]


def _attn_block(h, n1s, n1b, wdkv, wdq, wuk, wuv, wuq, wkr, wqr, wo):
    kc, kr, qc, qr, v = pl.pallas_call(
        _attn1_body,
        grid=(_NAB,),
        in_specs=[
            pl.BlockSpec((_AB, H_DIM), lambda r: (r, 0)),
            pl.BlockSpec((1, H_DIM), lambda r: (0, 0)),
            pl.BlockSpec((1, H_DIM), lambda r: (0, 0)),
            pl.BlockSpec((H_DIM, C_DIM), lambda r: (0, 0)),
            pl.BlockSpec((H_DIM, C_DIM), lambda r: (0, 0)),
            pl.BlockSpec((C_DIM, UP_DIM), lambda r: (0, 0)),
            pl.BlockSpec((C_DIM, H_DIM), lambda r: (0, 0)),
            pl.BlockSpec((C_DIM, UP_DIM), lambda r: (0, 0)),
            pl.BlockSpec((H_DIM, UP_DIM), lambda r: (0, 0)),
            pl.BlockSpec((C_DIM, UP_DIM), lambda r: (0, 0)),
            pl.BlockSpec((_AB, UP_DIM), lambda r: (r, 0)),
            pl.BlockSpec((_AB, UP_DIM), lambda r: (r, 0)),
            pl.BlockSpec((UP_DIM, UP_DIM), lambda r: (0, 0)),
        ],
        out_specs=[
            pl.BlockSpec((_AB, UP_DIM), lambda r: (r, 0)),
            pl.BlockSpec((_AB, UP_DIM), lambda r: (r, 0)),
            pl.BlockSpec((_AB, UP_DIM), lambda r: (r, 0)),
            pl.BlockSpec((_AB, UP_DIM), lambda r: (r, 0)),
            pl.BlockSpec((_AB, H_DIM), lambda r: (r, 0)),
        ],
        out_shape=[
            jax.ShapeDtypeStruct((SEQ, UP_DIM), jnp.float32),
            jax.ShapeDtypeStruct((SEQ, UP_DIM), jnp.float32),
            jax.ShapeDtypeStruct((SEQ, UP_DIM), jnp.float32),
            jax.ShapeDtypeStruct((SEQ, UP_DIM), jnp.float32),
            jax.ShapeDtypeStruct((SEQ, H_DIM), jnp.float32),
        ],
    )(h, n1s, n1b, wdkv, wdq, wuk, wuv, wuq, wkr, wqr, COSF, SINF, PMAT)
    attn = pl.pallas_call(
        _attn2_body,
        grid=(_NAB,),
        in_specs=[
            pl.BlockSpec((_AB, UP_DIM), lambda i: (i, 0)),
            pl.BlockSpec((_AB, UP_DIM), lambda i: (i, 0)),
            pl.BlockSpec((SEQ, UP_DIM), lambda i: (0, 0)),
            pl.BlockSpec((SEQ, UP_DIM), lambda i: (0, 0)),
            pl.BlockSpec((SEQ, H_DIM), lambda i: (0, 0)),
        ],
        out_specs=pl.BlockSpec((_AB, H_DIM), lambda i: (i, 0)),
        out_shape=jax.ShapeDtypeStruct((SEQ, H_DIM), jnp.float32),
    )(kc, kr, qc, qr, v)
    return pl.pallas_call(
        _attn3_body,
        grid=(_NAB,),
        in_specs=[
            pl.BlockSpec((_AB, H_DIM), lambda r: (r, 0)),
            pl.BlockSpec((_AB, H_DIM), lambda r: (r, 0)),
            pl.BlockSpec((H_DIM, H_DIM), lambda r: (0, 0)),
        ],
        out_specs=pl.BlockSpec((_AB, H_DIM), lambda r: (r, 0)),
        out_shape=jax.ShapeDtypeStruct((SEQ, H_DIM), jnp.float32),
    )(h, attn, wo)


_RB = 256
_NRB = SEQ // _RB


def _shared_body(a_ref, h_ref, n2s_ref, n2b_ref, up_ref, dn_ref, rt_ref,
                 y_ref, x_ref, w_ref):
    xx = _ln(a_ref[...], n2s_ref[...], n2b_ref[...])
    g = _gelu(_dot(xx, up_ref[...]))
    y_ref[...] = _dot(g, dn_ref[...]) + xx + h_ref[...]
    x_ref[...] = xx
    logits = _dot(xx, rt_ref[...])
    p = jnp.exp(logits - jnp.max(logits, axis=1, keepdims=True))
    p = p / jnp.sum(p, axis=1, keepdims=True)
    m1 = jnp.max(p, axis=1, keepdims=True)
    p_wo1 = jnp.where(p >= m1, -jnp.inf, p)
    m2 = jnp.max(p_wo1, axis=1, keepdims=True)
    w_ref[...] = jnp.where(p >= m2, p, 0.0)


def _shared_and_router(a, h, n2s, n2b, sh_up, sh_down, router):
    return pl.pallas_call(
        _shared_body,
        grid=(_NRB,),
        in_specs=[
            pl.BlockSpec((_RB, H_DIM), lambda r: (r, 0)),
            pl.BlockSpec((_RB, H_DIM), lambda r: (r, 0)),
            pl.BlockSpec((1, H_DIM), lambda r: (0, 0)),
            pl.BlockSpec((1, H_DIM), lambda r: (0, 0)),
            pl.BlockSpec((H_DIM, E_DIM), lambda r: (0, 0)),
            pl.BlockSpec((E_DIM, H_DIM), lambda r: (0, 0)),
            pl.BlockSpec((H_DIM, N_ROUTED), lambda r: (0, 0)),
        ],
        out_specs=[
            pl.BlockSpec((_RB, H_DIM), lambda r: (r, 0)),
            pl.BlockSpec((_RB, H_DIM), lambda r: (r, 0)),
            pl.BlockSpec((_RB, N_ROUTED), lambda r: (r, 0)),
        ],
        out_shape=[
            jax.ShapeDtypeStruct((SEQ, H_DIM), jnp.float32),
            jax.ShapeDtypeStruct((SEQ, H_DIM), jnp.float32),
            jax.ShapeDtypeStruct((SEQ, N_ROUTED), jnp.float32),
        ],
    )(a, h, n2s, n2b, sh_up, sh_down, router)


N_PAIRS = TOP_K * SEQ


def _route_meta_body(w_ref, inv1_ref, inv2_ref, wt1_ref, wt2_ref, offs_ref):
    w = w_ref[...]
    abool = w > 0.0
    ai = abool.astype(jnp.int32)
    c = ai
    for k in (1, 2, 4):
        c = c + jnp.pad(c, ((0, 0), (k, 0)))[:, :N_ROUTED]
    first = jnp.logical_and(abool, c == 1)
    second = jnp.logical_and(abool, c == 2)
    m2 = jnp.logical_or(first, second).astype(jnp.int32)
    cnt = jnp.sum(m2, axis=0, keepdims=True)
    oc = cnt
    for k in (1, 2, 4):
        oc = oc + jnp.pad(oc, ((0, 0), (k, 0)))[:, :N_ROUTED]
    offs = oc - cnt
    rk = m2
    k = 1
    while k < SEQ:
        rk = rk + jnp.pad(rk, ((k, 0), (0, 0)))[:SEQ, :]
        k *= 2
    pos = offs + (rk - m2)
    fi = first.astype(jnp.int32)
    si = second.astype(jnp.int32)
    inv1_ref[...] = jnp.sum(pos * fi, axis=1, keepdims=True)
    inv2_ref[...] = jnp.sum(pos * si, axis=1, keepdims=True)
    wt1_ref[...] = jnp.sum(w * fi.astype(jnp.float32), axis=1, keepdims=True)
    wt2_ref[...] = jnp.sum(w * si.astype(jnp.float32), axis=1, keepdims=True)
    offs_ref[...] = offs


def _route_meta(w):
    return pl.pallas_call(
        _route_meta_body,
        out_shape=[
            jax.ShapeDtypeStruct((SEQ, 1), jnp.int32),
            jax.ShapeDtypeStruct((SEQ, 1), jnp.int32),
            jax.ShapeDtypeStruct((SEQ, 1), jnp.float32),
            jax.ShapeDtypeStruct((SEQ, 1), jnp.float32),
            jax.ShapeDtypeStruct((1, N_ROUTED), jnp.int32),
        ],
    )(w)


def _sc_scatter_x(xx, idx3):
    mesh = plsc.VectorSubcoreMesh(core_axis_name="c", subcore_axis_name="s")

    @functools.partial(
        pl.kernel, mesh=mesh,
        out_type=jax.ShapeDtypeStruct((N_PAIRS, H_DIM), jnp.float32),
        scratch_types=[
            pltpu.VMEM((TOP_K, _B_PER_W), jnp.int32),
            pltpu.VMEM((_B_PER_W, H_DIM), jnp.float32),
        ],
    )
    def k(xx_hbm, idx_hbm, out_hbm, idx_v, rows_v):
        wid = lax.axis_index("s") * 2 + lax.axis_index("c")
        base = wid * _B_PER_W
        pltpu.sync_copy(idx_hbm.at[wid], idx_v)
        pltpu.sync_copy(xx_hbm.at[pl.ds(base, _B_PER_W)], rows_v)
        pltpu.sync_copy(rows_v, out_hbm.at[idx_v.at[0]])
        pltpu.sync_copy(rows_v, out_hbm.at[idx_v.at[1]])

    return k(xx, idx3)


def _sc_gather2(os_, idx3):
    mesh = plsc.VectorSubcoreMesh(core_axis_name="c", subcore_axis_name="s")

    @functools.partial(
        pl.kernel, mesh=mesh,
        out_type=[
            jax.ShapeDtypeStruct((SEQ, H_DIM), jnp.float32),
            jax.ShapeDtypeStruct((SEQ, H_DIM), jnp.float32),
        ],
        scratch_types=[
            pltpu.VMEM((TOP_K, _B_PER_W), jnp.int32),
            pltpu.VMEM((_B_PER_W, H_DIM), jnp.float32),
        ],
    )
    def k(os_hbm, idx_hbm, g1_hbm, g2_hbm, idx_v, rows_v):
        wid = lax.axis_index("s") * 2 + lax.axis_index("c")
        base = wid * _B_PER_W
        pltpu.sync_copy(idx_hbm.at[wid], idx_v)
        pltpu.sync_copy(os_hbm.at[idx_v.at[0]], rows_v)
        pltpu.sync_copy(rows_v, g1_hbm.at[pl.ds(base, _B_PER_W)])
        pltpu.sync_copy(os_hbm.at[idx_v.at[1]], rows_v)
        pltpu.sync_copy(rows_v, g2_hbm.at[pl.ds(base, _B_PER_W)])

    return k(os_, idx3)


_TB = 128
_NT = N_PAIRS // _TB
_NWORK = _NT + N_ROUTED - 1


def _etile_body(meta_ref, x_ref, up_ref, dn_ref, out_ref):
    wi = pl.program_id(0)
    tile = meta_ref[0, wi]
    start = meta_ref[2, wi]
    end = meta_ref[3, wi]
    fst = meta_ref[4, wi]
    rows = tile * _TB + lax.broadcasted_iota(jnp.int32, (_TB, 1), 0)
    m = jnp.logical_and(rows >= start, rows < end)
    xm = jnp.where(m, x_ref[...], 0.0)
    t = _dot(_gelu(_dot(xm, up_ref[0])), dn_ref[0])

    @pl.when(fst == 1)
    def _():
        out_ref[...] = t

    @pl.when(fst == 0)
    def _():
        out_ref[...] = out_ref[...] + t


def _expert_tiles(meta, xs, r_up, r_down):
    grid_spec = pltpu.PrefetchScalarGridSpec(
        num_scalar_prefetch=1,
        grid=(_NWORK,),
        in_specs=[
            pl.BlockSpec((_TB, H_DIM), lambda wi, m: (m[0, wi], 0)),
            pl.BlockSpec((1, H_DIM, E_DIM), lambda wi, m: (m[1, wi], 0, 0)),
            pl.BlockSpec((1, E_DIM, H_DIM), lambda wi, m: (m[1, wi], 0, 0)),
        ],
        out_specs=pl.BlockSpec((_TB, H_DIM), lambda wi, m: (m[0, wi], 0)),
    )
    return pl.pallas_call(
        _etile_body,
        grid_spec=grid_spec,
        out_shape=jax.ShapeDtypeStruct((N_PAIRS, H_DIM), jnp.float32),
    )(meta, xs, r_up, r_down)


def _combine_body(y_ref, g1_ref, g2_ref, wt1_ref, wt2_ref, nfs_ref, nfb_ref,
                  out_ref):
    routed = g1_ref[...] * wt1_ref[...] + g2_ref[...] * wt2_ref[...]
    out_ref[...] = _ln(routed + y_ref[...], nfs_ref[...], nfb_ref[...])


def _combine_final_ln(y, g1, g2, wt1, wt2, nfs, nfb):
    return pl.pallas_call(
        _combine_body,
        grid=(_NRB,),
        in_specs=[
            pl.BlockSpec((_RB, H_DIM), lambda r: (r, 0)),
            pl.BlockSpec((_RB, H_DIM), lambda r: (r, 0)),
            pl.BlockSpec((_RB, H_DIM), lambda r: (r, 0)),
            pl.BlockSpec((_RB, 1), lambda r: (r, 0)),
            pl.BlockSpec((_RB, 1), lambda r: (r, 0)),
            pl.BlockSpec((1, H_DIM), lambda r: (0, 0)),
            pl.BlockSpec((1, H_DIM), lambda r: (0, 0)),
        ],
        out_specs=pl.BlockSpec((_RB, H_DIM), lambda r: (r, 0)),
        out_shape=jax.ShapeDtypeStruct((SEQ, H_DIM), jnp.float32),
    )(y, g1, g2, wt1, wt2, nfs, nfb)


_CB = 3200
_NCB = N_TOKENS // _CB


def _head_body(mf_ref, proj_ref, out_ref):
    out_ref[...] = _dot(mf_ref[...], proj_ref[...])


_HRB = 1024
_NHRB = SEQ // _HRB


def _lm_head(mf, proj):
    return pl.pallas_call(
        _head_body,
        grid=(_NCB, _NHRB),
        in_specs=[
            pl.BlockSpec((_HRB, H_DIM), lambda c, r: (r, 0)),
            pl.BlockSpec((H_DIM, _CB), lambda c, r: (0, c)),
        ],
        out_specs=pl.BlockSpec((_HRB, _CB), lambda c, r: (r, c)),
        out_shape=jax.ShapeDtypeStruct((SEQ, N_TOKENS), jnp.float32),
    )(mf, proj)


def kernel(x, emb, n1_scale, n1_bias, W_dkv, W_dq, W_uk, W_uv, W_uq, W_kr,
           W_qr, W_o, n2_scale, n2_bias, sh_up, sh_down, r_up, r_down,
           router, nf_scale, nf_bias, proj):
    idx = x.reshape(SEQ)
    h = _emb_gather(emb, idx)
    n1s = n1_scale.reshape(1, H_DIM)
    n1b = n1_bias.reshape(1, H_DIM)
    a = _attn_block(h, n1s, n1b, W_dkv, W_dq, W_uk, W_uv, W_uq, W_kr, W_qr,
                    W_o)
    y, xx, w = _shared_and_router(a, h, n2_scale.reshape(1, H_DIM),
                                  n2_bias.reshape(1, H_DIM), sh_up, sh_down,
                                  router)
    inv1, inv2, wt1, wt2, offs = _route_meta(w)
    offs9 = jnp.concatenate([offs.reshape(N_ROUTED),
                             jnp.array([N_PAIRS], jnp.int32)])
    tau0 = jnp.arange(_NT, dtype=jnp.int32) * _TB
    flag = jnp.logical_and(offs9[None, :N_ROUTED] < (tau0 + _TB)[:, None],
                           offs9[None, 1:] > tau0[:, None])
    flat = flag.reshape(_NT * N_ROUTED)
    order = jnp.argsort(jnp.logical_not(flat), stable=True).astype(jnp.int32)
    sel = order[:_NWORK]
    nact = jnp.sum(flat.astype(jnp.int32))
    valid = jnp.arange(_NWORK, dtype=jnp.int32) < nact
    tile_w = jnp.where(valid, sel // N_ROUTED, _NT - 1)
    e_w = jnp.where(valid, sel % N_ROUTED, N_ROUTED - 1)
    s_w = jnp.where(valid, jnp.take(offs9, e_w), 0)
    en_w = jnp.where(valid, jnp.take(offs9, e_w + 1), 0)
    fst_w = jnp.where(
        valid,
        jnp.concatenate([jnp.array([1], jnp.int32),
                         (tile_w[1:] != tile_w[:-1]).astype(jnp.int32)]),
        0)
    meta = jnp.stack([tile_w, e_w, s_w, en_w, fst_w]).astype(jnp.int32)
    idx3 = jnp.stack([inv1.reshape(_NW, _B_PER_W),
                      inv2.reshape(_NW, _B_PER_W)], axis=1)
    xs = _sc_scatter_x(xx, idx3)
    os_ = _expert_tiles(meta, xs, r_up, r_down)
    g1, g2 = _sc_gather2(os_, idx3)
    mf = _combine_final_ln(y, g1, g2, wt1, wt2, nf_scale.reshape(1, H_DIM),
                           nf_bias.reshape(1, H_DIM))
    logits = _lm_head(mf, proj)
    return logits[None]


# speedup vs baseline: 1.1592x; 1.0201x over previous
import functools
import math

import jax
import jax.numpy as jnp
import numpy as np
from jax import lax
from jax.experimental import pallas as pl
from jax.experimental.pallas import tpu as pltpu
from jax.experimental.pallas import tpu_sc as plsc

H_DIM = 768
N_HEADS = 12
C_DIM = 256
N_TOKENS = 32000
N_ROUTED = 8
TOP_K = 2
E_DIM = 4 * H_DIM
D_HEAD = H_DIM // N_HEADS
D_ROPE = D_HEAD // 2
UP_DIM = (D_HEAD - D_ROPE) * N_HEADS
SEQ = 2048

PREC = jax.lax.Precision.DEFAULT

_half = D_ROPE // 2
_inv_freq = 1.0 / (10000.0 ** (np.arange(0, _half, 2, dtype=np.float32) / _half))
_t = np.arange(SEQ, dtype=np.float32) / 40.0
_freqs = _t[:, None] * _inv_freq[None, :]
_c8 = np.cos(_freqs)
_s8 = np.sin(_freqs)
_ones16 = np.ones((SEQ, _half), np.float32)
_zeros16 = np.zeros((SEQ, _half), np.float32)
_COSF = np.tile(np.concatenate([_c8, _c8, _ones16], axis=1), (1, N_HEADS))
_SINF = np.tile(np.concatenate([_s8, _s8, _zeros16], axis=1), (1, N_HEADS))
_P = np.zeros((UP_DIM, UP_DIM), np.float32)
for _h in range(N_HEADS):
    _b = _h * D_ROPE
    for _i in range(8):
        _P[_b + 8 + _i, _b + _i] = -1.0
        _P[_b + _i, _b + 8 + _i] = 1.0
COSF = _COSF
SINF = _SINF
PMAT = _P


def _dot(a, b):
    return lax.dot_general(a, b, (((1,), (0,)), ((), ())),
                           preferred_element_type=jnp.float32, precision=PREC)


def _dot_fast(a, b):
    return lax.dot_general(a.astype(jnp.bfloat16), b.astype(jnp.bfloat16),
                           (((1,), (0,)), ((), ())),
                           preferred_element_type=jnp.float32)


def _dot_x(a, b):
    return lax.dot_general(a, b, (((1,), (0,)), ((), ())),
                           preferred_element_type=jnp.float32,
                           precision=jax.lax.Precision.HIGHEST)


def _ln(x, scale, bias):
    m = jnp.mean(x, axis=-1, keepdims=True)
    d = x - m
    v = jnp.mean(d * d, axis=-1, keepdims=True)
    return d * lax.rsqrt(v + 1e-5) * scale + bias


def _erf(x):
    ax = jnp.abs(x)
    t = 1.0 / (1.0 + 0.3275911 * ax)
    y = ((((1.061405429 * t - 1.453152027) * t + 1.421413741) * t
          - 0.284496736) * t + 0.254829592) * t
    y = 1.0 - y * jnp.exp(-ax * ax)
    return jnp.sign(x) * y


def _gelu(x):
    return 0.5 * x * (1.0 + _erf(x * (1.0 / math.sqrt(2.0))))


_NW = 32
_B_PER_W = SEQ // _NW


def _emb_gather(emb, idx):
    mesh = plsc.VectorSubcoreMesh(core_axis_name="c", subcore_axis_name="s")

    @functools.partial(
        pl.kernel, mesh=mesh,
        out_type=jax.ShapeDtypeStruct((SEQ, H_DIM), jnp.float32),
        scratch_types=[
            pltpu.VMEM((_B_PER_W,), jnp.int32),
            pltpu.VMEM((_B_PER_W, H_DIM), jnp.float32),
            pltpu.SemaphoreType.DMA,
        ],
    )
    def k(emb_hbm, idx_hbm, out_hbm, idx_v, rows_v, sem):
        wid = lax.axis_index("s") * 2 + lax.axis_index("c")
        base = wid * _B_PER_W
        pltpu.sync_copy(idx_hbm.at[pl.ds(base, _B_PER_W)], idx_v)
        pltpu.async_copy(emb_hbm.at[idx_v], rows_v, sem).wait()
        pltpu.sync_copy(rows_v, out_hbm.at[pl.ds(base, _B_PER_W)])

    return k(emb, idx)


_AB = 512
_NAB = SEQ // _AB


def _attn1_body(h_ref, n1s_ref, n1b_ref, wdkv_ref, wdq_ref, wuk_ref,
                wuv_ref, wuq_ref, wkr_ref, wqr_ref, cos_ref, sin_ref,
                p_ref, kc_ref, kr_ref, qc_ref, qr_ref, v_ref):
    h1 = _ln(h_ref[...], n1s_ref[...], n1b_ref[...])
    c_kv = _dot(h1, wdkv_ref[...])
    c_q = _dot(h1, wdq_ref[...])
    qc_ref[...] = _dot(c_q, wuq_ref[...])
    kc_ref[...] = _dot(c_kv, wuk_ref[...])
    v_ref[...] = _dot(c_kv, wuv_ref[...])
    k_r = _dot(h1, wkr_ref[...])
    q_r = _dot(c_q, wqr_ref[...])
    cos = cos_ref[...]
    sin = sin_ref[...]
    pm = p_ref[...]
    kr_ref[...] = k_r * cos + _dot_x(k_r, pm) * sin
    qr_ref[...] = q_r * cos + _dot_x(q_r, pm) * sin


def _attn2_body(kc_ref, kr_ref, qc_ref, qr_ref, v_ref, out_ref):
    kc = kc_ref[...]
    kr = kr_ref[...]
    qc = qc_ref[...]
    qr = qr_ref[...]
    v = v_ref[...]
    outs = []
    for hh in range(N_HEADS):
        s32 = slice(hh * D_ROPE, (hh + 1) * D_ROPE)
        s64 = slice(hh * D_HEAD, (hh + 1) * D_HEAD)
        k_cat = jnp.concatenate([kc[:, s32], kr[:, s32]], axis=1)
        q_cat = jnp.concatenate([qc[:, s32], qr[:, s32]], axis=1)
        s = lax.dot_general(k_cat, q_cat, (((1,), (1,)), ((), ())),
                            preferred_element_type=jnp.float32,
                            precision=PREC) * (1.0 / math.sqrt(D_HEAD))
        outs.append(_dot(s, v[:, s64]))
    out_ref[...] = jnp.concatenate(outs, axis=1)


def _attn3_body(h_ref, attn_ref, wo_ref, out_ref):
    out_ref[...] = _dot(attn_ref[...], wo_ref[...]) + h_ref[...]


def _attn_block(h, n1s, n1b, wdkv, wdq, wuk, wuv, wuq, wkr, wqr, wo):
    kc, kr, qc, qr, v = pl.pallas_call(
        _attn1_body,
        grid=(_NAB,),
        in_specs=[
            pl.BlockSpec((_AB, H_DIM), lambda r: (r, 0)),
            pl.BlockSpec((1, H_DIM), lambda r: (0, 0)),
            pl.BlockSpec((1, H_DIM), lambda r: (0, 0)),
            pl.BlockSpec((H_DIM, C_DIM), lambda r: (0, 0)),
            pl.BlockSpec((H_DIM, C_DIM), lambda r: (0, 0)),
            pl.BlockSpec((C_DIM, UP_DIM), lambda r: (0, 0)),
            pl.BlockSpec((C_DIM, H_DIM), lambda r: (0, 0)),
            pl.BlockSpec((C_DIM, UP_DIM), lambda r: (0, 0)),
            pl.BlockSpec((H_DIM, UP_DIM), lambda r: (0, 0)),
            pl.BlockSpec((C_DIM, UP_DIM), lambda r: (0, 0)),
            pl.BlockSpec((_AB, UP_DIM), lambda r: (r, 0)),
            pl.BlockSpec((_AB, UP_DIM), lambda r: (r, 0)),
            pl.BlockSpec((UP_DIM, UP_DIM), lambda r: (0, 0)),
        ],
        out_specs=[
            pl.BlockSpec((_AB, UP_DIM), lambda r: (r, 0)),
            pl.BlockSpec((_AB, UP_DIM), lambda r: (r, 0)),
            pl.BlockSpec((_AB, UP_DIM), lambda r: (r, 0)),
            pl.BlockSpec((_AB, UP_DIM), lambda r: (r, 0)),
            pl.BlockSpec((_AB, H_DIM), lambda r: (r, 0)),
        ],
        out_shape=[
            jax.ShapeDtypeStruct((SEQ, UP_DIM), jnp.float32),
            jax.ShapeDtypeStruct((SEQ, UP_DIM), jnp.float32),
            jax.ShapeDtypeStruct((SEQ, UP_DIM), jnp.float32),
            jax.ShapeDtypeStruct((SEQ, UP_DIM), jnp.float32),
            jax.ShapeDtypeStruct((SEQ, H_DIM), jnp.float32),
        ],
    )(h, n1s, n1b, wdkv, wdq, wuk, wuv, wuq, wkr, wqr, COSF, SINF, PMAT)
    attn = pl.pallas_call(
        _attn2_body,
        grid=(_NAB,),
        in_specs=[
            pl.BlockSpec((_AB, UP_DIM), lambda i: (i, 0)),
            pl.BlockSpec((_AB, UP_DIM), lambda i: (i, 0)),
            pl.BlockSpec((SEQ, UP_DIM), lambda i: (0, 0)),
            pl.BlockSpec((SEQ, UP_DIM), lambda i: (0, 0)),
            pl.BlockSpec((SEQ, H_DIM), lambda i: (0, 0)),
        ],
        out_specs=pl.BlockSpec((_AB, H_DIM), lambda i: (i, 0)),
        out_shape=jax.ShapeDtypeStruct((SEQ, H_DIM), jnp.float32),
    )(kc, kr, qc, qr, v)
    return pl.pallas_call(
        _attn3_body,
        grid=(_NAB,),
        in_specs=[
            pl.BlockSpec((_AB, H_DIM), lambda r: (r, 0)),
            pl.BlockSpec((_AB, H_DIM), lambda r: (r, 0)),
            pl.BlockSpec((H_DIM, H_DIM), lambda r: (0, 0)),
        ],
        out_specs=pl.BlockSpec((_AB, H_DIM), lambda r: (r, 0)),
        out_shape=jax.ShapeDtypeStruct((SEQ, H_DIM), jnp.float32),
    )(h, attn, wo)


_RB = 512
_NRB = SEQ // _RB


def _shared_body(a_ref, h_ref, n2s_ref, n2b_ref, up_ref, dn_ref, rt_ref,
                 y_ref, x_ref, w_ref):
    xx = _ln(a_ref[...], n2s_ref[...], n2b_ref[...])
    g = _gelu(_dot(xx, up_ref[...]))
    y_ref[...] = _dot(g, dn_ref[...]) + xx + h_ref[...]
    x_ref[...] = xx
    logits = _dot(xx, rt_ref[...])
    p = jnp.exp(logits - jnp.max(logits, axis=1, keepdims=True))
    p = p / jnp.sum(p, axis=1, keepdims=True)
    m1 = jnp.max(p, axis=1, keepdims=True)
    p_wo1 = jnp.where(p >= m1, -jnp.inf, p)
    m2 = jnp.max(p_wo1, axis=1, keepdims=True)
    w_ref[...] = jnp.where(p >= m2, p, 0.0)


def _shared_and_router(a, h, n2s, n2b, sh_up, sh_down, router):
    return pl.pallas_call(
        _shared_body,
        grid=(_NRB,),
        in_specs=[
            pl.BlockSpec((_RB, H_DIM), lambda r: (r, 0)),
            pl.BlockSpec((_RB, H_DIM), lambda r: (r, 0)),
            pl.BlockSpec((1, H_DIM), lambda r: (0, 0)),
            pl.BlockSpec((1, H_DIM), lambda r: (0, 0)),
            pl.BlockSpec((H_DIM, E_DIM), lambda r: (0, 0)),
            pl.BlockSpec((E_DIM, H_DIM), lambda r: (0, 0)),
            pl.BlockSpec((H_DIM, N_ROUTED), lambda r: (0, 0)),
        ],
        out_specs=[
            pl.BlockSpec((_RB, H_DIM), lambda r: (r, 0)),
            pl.BlockSpec((_RB, H_DIM), lambda r: (r, 0)),
            pl.BlockSpec((_RB, N_ROUTED), lambda r: (r, 0)),
        ],
        out_shape=[
            jax.ShapeDtypeStruct((SEQ, H_DIM), jnp.float32),
            jax.ShapeDtypeStruct((SEQ, H_DIM), jnp.float32),
            jax.ShapeDtypeStruct((SEQ, N_ROUTED), jnp.float32),
        ],
    )(a, h, n2s, n2b, sh_up, sh_down, router)


N_PAIRS = TOP_K * SEQ


def _route_meta_body(w_ref, inv1_ref, inv2_ref, wt1_ref, wt2_ref, offs_ref):
    w = w_ref[...]
    abool = w > 0.0
    ai = abool.astype(jnp.int32)
    c = ai
    for k in (1, 2, 4):
        c = c + jnp.pad(c, ((0, 0), (k, 0)))[:, :N_ROUTED]
    first = jnp.logical_and(abool, c == 1)
    second = jnp.logical_and(abool, c == 2)
    m2 = jnp.logical_or(first, second).astype(jnp.int32)
    cnt = jnp.sum(m2, axis=0, keepdims=True)
    oc = cnt
    for k in (1, 2, 4):
        oc = oc + jnp.pad(oc, ((0, 0), (k, 0)))[:, :N_ROUTED]
    offs = oc - cnt
    rk = m2
    k = 1
    while k < SEQ:
        rk = rk + jnp.pad(rk, ((k, 0), (0, 0)))[:SEQ, :]
        k *= 2
    pos = offs + (rk - m2)
    fi = first.astype(jnp.int32)
    si = second.astype(jnp.int32)
    inv1_ref[...] = jnp.sum(pos * fi, axis=1, keepdims=True)
    inv2_ref[...] = jnp.sum(pos * si, axis=1, keepdims=True)
    wt1_ref[...] = jnp.sum(w * fi.astype(jnp.float32), axis=1, keepdims=True)
    wt2_ref[...] = jnp.sum(w * si.astype(jnp.float32), axis=1, keepdims=True)
    offs_ref[...] = offs


def _route_meta(w):
    return pl.pallas_call(
        _route_meta_body,
        out_shape=[
            jax.ShapeDtypeStruct((SEQ, 1), jnp.int32),
            jax.ShapeDtypeStruct((SEQ, 1), jnp.int32),
            jax.ShapeDtypeStruct((SEQ, 1), jnp.float32),
            jax.ShapeDtypeStruct((SEQ, 1), jnp.float32),
            jax.ShapeDtypeStruct((1, N_ROUTED), jnp.int32),
        ],
    )(w)


def _sc_scatter_x(xx, idx3):
    mesh = plsc.VectorSubcoreMesh(core_axis_name="c", subcore_axis_name="s")

    @functools.partial(
        pl.kernel, mesh=mesh,
        out_type=jax.ShapeDtypeStruct((N_PAIRS, H_DIM), jnp.float32),
        scratch_types=[
            pltpu.VMEM((TOP_K, _B_PER_W), jnp.int32),
            pltpu.VMEM((_B_PER_W, H_DIM), jnp.float32),
        ],
    )
    def k(xx_hbm, idx_hbm, out_hbm, idx_v, rows_v):
        wid = lax.axis_index("s") * 2 + lax.axis_index("c")
        base = wid * _B_PER_W
        pltpu.sync_copy(idx_hbm.at[wid], idx_v)
        pltpu.sync_copy(xx_hbm.at[pl.ds(base, _B_PER_W)], rows_v)
        pltpu.sync_copy(rows_v, out_hbm.at[idx_v.at[0]])
        pltpu.sync_copy(rows_v, out_hbm.at[idx_v.at[1]])

    return k(xx, idx3)


def _sc_gather2(os_, idx3):
    mesh = plsc.VectorSubcoreMesh(core_axis_name="c", subcore_axis_name="s")

    @functools.partial(
        pl.kernel, mesh=mesh,
        out_type=[
            jax.ShapeDtypeStruct((SEQ, H_DIM), jnp.float32),
            jax.ShapeDtypeStruct((SEQ, H_DIM), jnp.float32),
        ],
        scratch_types=[
            pltpu.VMEM((TOP_K, _B_PER_W), jnp.int32),
            pltpu.VMEM((_B_PER_W, H_DIM), jnp.float32),
        ],
    )
    def k(os_hbm, idx_hbm, g1_hbm, g2_hbm, idx_v, rows_v):
        wid = lax.axis_index("s") * 2 + lax.axis_index("c")
        base = wid * _B_PER_W
        pltpu.sync_copy(idx_hbm.at[wid], idx_v)
        pltpu.sync_copy(os_hbm.at[idx_v.at[0]], rows_v)
        pltpu.sync_copy(rows_v, g1_hbm.at[pl.ds(base, _B_PER_W)])
        pltpu.sync_copy(os_hbm.at[idx_v.at[1]], rows_v)
        pltpu.sync_copy(rows_v, g2_hbm.at[pl.ds(base, _B_PER_W)])

    return k(os_, idx3)


_TB = 128
_NT = N_PAIRS // _TB
_NWORK = _NT + N_ROUTED - 1


def _etile_body(meta_ref, x_ref, up_ref, dn_ref, out_ref):
    wi = pl.program_id(0)
    tile = meta_ref[0, wi]
    start = meta_ref[2, wi]
    end = meta_ref[3, wi]
    fst = meta_ref[4, wi]
    rows = tile * _TB + lax.broadcasted_iota(jnp.int32, (_TB, 1), 0)
    m = jnp.logical_and(rows >= start, rows < end)
    xm = jnp.where(m, x_ref[...], 0.0)
    t = _dot(_gelu(_dot(xm, up_ref[0])), dn_ref[0])

    @pl.when(fst == 1)
    def _():
        out_ref[...] = t

    @pl.when(fst == 0)
    def _():
        out_ref[...] = out_ref[...] + t


def _expert_tiles(meta, xs, r_up, r_down):
    grid_spec = pltpu.PrefetchScalarGridSpec(
        num_scalar_prefetch=1,
        grid=(_NWORK,),
        in_specs=[
            pl.BlockSpec((_TB, H_DIM), lambda wi, m: (m[0, wi], 0)),
            pl.BlockSpec((1, H_DIM, E_DIM), lambda wi, m: (m[1, wi], 0, 0)),
            pl.BlockSpec((1, E_DIM, H_DIM), lambda wi, m: (m[1, wi], 0, 0)),
        ],
        out_specs=pl.BlockSpec((_TB, H_DIM), lambda wi, m: (m[0, wi], 0)),
    )
    return pl.pallas_call(
        _etile_body,
        grid_spec=grid_spec,
        out_shape=jax.ShapeDtypeStruct((N_PAIRS, H_DIM), jnp.float32),
    )(meta, xs, r_up, r_down)


def _combine_body(y_ref, g1_ref, g2_ref, wt1_ref, wt2_ref, nfs_ref, nfb_ref,
                  out_ref):
    routed = g1_ref[...] * wt1_ref[...] + g2_ref[...] * wt2_ref[...]
    out_ref[...] = _ln(routed + y_ref[...], nfs_ref[...], nfb_ref[...])


def _combine_final_ln(y, g1, g2, wt1, wt2, nfs, nfb):
    return pl.pallas_call(
        _combine_body,
        grid=(_NRB,),
        in_specs=[
            pl.BlockSpec((_RB, H_DIM), lambda r: (r, 0)),
            pl.BlockSpec((_RB, H_DIM), lambda r: (r, 0)),
            pl.BlockSpec((_RB, H_DIM), lambda r: (r, 0)),
            pl.BlockSpec((_RB, 1), lambda r: (r, 0)),
            pl.BlockSpec((_RB, 1), lambda r: (r, 0)),
            pl.BlockSpec((1, H_DIM), lambda r: (0, 0)),
            pl.BlockSpec((1, H_DIM), lambda r: (0, 0)),
        ],
        out_specs=pl.BlockSpec((_RB, H_DIM), lambda r: (r, 0)),
        out_shape=jax.ShapeDtypeStruct((SEQ, H_DIM), jnp.float32),
    )(y, g1, g2, wt1, wt2, nfs, nfb)


_CB = 3200
_NCB = N_TOKENS // _CB


def _head_body(mf_ref, proj_ref, out_ref):
    out_ref[...] = _dot(mf_ref[...], proj_ref[...])


_HRB = 1024
_NHRB = SEQ // _HRB


def _lm_head(mf, proj):
    return pl.pallas_call(
        _head_body,
        grid=(_NCB, _NHRB),
        in_specs=[
            pl.BlockSpec((_HRB, H_DIM), lambda c, r: (r, 0)),
            pl.BlockSpec((H_DIM, _CB), lambda c, r: (0, c)),
        ],
        out_specs=pl.BlockSpec((_HRB, _CB), lambda c, r: (r, c)),
        out_shape=jax.ShapeDtypeStruct((SEQ, N_TOKENS), jnp.float32),
    )(mf, proj)


def kernel(x, emb, n1_scale, n1_bias, W_dkv, W_dq, W_uk, W_uv, W_uq, W_kr,
           W_qr, W_o, n2_scale, n2_bias, sh_up, sh_down, r_up, r_down,
           router, nf_scale, nf_bias, proj):
    idx = x.reshape(SEQ)
    h = _emb_gather(emb, idx)
    n1s = n1_scale.reshape(1, H_DIM)
    n1b = n1_bias.reshape(1, H_DIM)
    a = _attn_block(h, n1s, n1b, W_dkv, W_dq, W_uk, W_uv, W_uq, W_kr, W_qr,
                    W_o)
    y, xx, w = _shared_and_router(a, h, n2_scale.reshape(1, H_DIM),
                                  n2_bias.reshape(1, H_DIM), sh_up, sh_down,
                                  router)
    inv1, inv2, wt1, wt2, offs = _route_meta(w)
    offs9 = jnp.concatenate([offs.reshape(N_ROUTED),
                             jnp.array([N_PAIRS], jnp.int32)])
    tau0 = jnp.arange(_NT, dtype=jnp.int32) * _TB
    flag = jnp.logical_and(offs9[None, :N_ROUTED] < (tau0 + _TB)[:, None],
                           offs9[None, 1:] > tau0[:, None])
    flat = flag.reshape(_NT * N_ROUTED)
    order = jnp.argsort(jnp.logical_not(flat), stable=True).astype(jnp.int32)
    sel = order[:_NWORK]
    nact = jnp.sum(flat.astype(jnp.int32))
    valid = jnp.arange(_NWORK, dtype=jnp.int32) < nact
    tile_w = jnp.where(valid, sel // N_ROUTED, _NT - 1)
    e_w = jnp.where(valid, sel % N_ROUTED, N_ROUTED - 1)
    s_w = jnp.where(valid, jnp.take(offs9, e_w), 0)
    en_w = jnp.where(valid, jnp.take(offs9, e_w + 1), 0)
    fst_w = jnp.where(
        valid,
        jnp.concatenate([jnp.array([1], jnp.int32),
                         (tile_w[1:] != tile_w[:-1]).astype(jnp.int32)]),
        0)
    meta = jnp.stack([tile_w, e_w, s_w, en_w, fst_w]).astype(jnp.int32)
    idx3 = jnp.stack([inv1.reshape(_NW, _B_PER_W),
                      inv2.reshape(_NW, _B_PER_W)], axis=1)
    xs = _sc_scatter_x(xx, idx3)
    os_ = _expert_tiles(meta, xs, r_up, r_down)
    g1, g2 = _sc_gather2(os_, idx3)
    mf = _combine_final_ln(y, g1, g2, wt1, wt2, nf_scale.reshape(1, H_DIM),
                           nf_bias.reshape(1, H_DIM))
    logits = _lm_head(mf, proj)
    return logits[None]


# speedup vs baseline: 1.1620x; 1.0025x over previous
import functools
import math

import jax
import jax.numpy as jnp
import numpy as np
from jax import lax
from jax.experimental import pallas as pl
from jax.experimental.pallas import tpu as pltpu
from jax.experimental.pallas import tpu_sc as plsc

H_DIM = 768
N_HEADS = 12
C_DIM = 256
N_TOKENS = 32000
N_ROUTED = 8
TOP_K = 2
E_DIM = 4 * H_DIM
D_HEAD = H_DIM // N_HEADS
D_ROPE = D_HEAD // 2
UP_DIM = (D_HEAD - D_ROPE) * N_HEADS
SEQ = 2048

PREC = jax.lax.Precision.DEFAULT

_half = D_ROPE // 2
_inv_freq = 1.0 / (10000.0 ** (np.arange(0, _half, 2, dtype=np.float32) / _half))
_t = np.arange(SEQ, dtype=np.float32) / 40.0
_freqs = _t[:, None] * _inv_freq[None, :]
_c8 = np.cos(_freqs)
_s8 = np.sin(_freqs)
_ones16 = np.ones((SEQ, _half), np.float32)
_zeros16 = np.zeros((SEQ, _half), np.float32)
_COSF = np.tile(np.concatenate([_c8, _c8, _ones16], axis=1), (1, N_HEADS))
_SINF = np.tile(np.concatenate([_s8, _s8, _zeros16], axis=1), (1, N_HEADS))
_P = np.zeros((UP_DIM, UP_DIM), np.float32)
for _h in range(N_HEADS):
    _b = _h * D_ROPE
    for _i in range(8):
        _P[_b + 8 + _i, _b + _i] = -1.0
        _P[_b + _i, _b + 8 + _i] = 1.0
COSF = _COSF
SINF = _SINF
PMAT = _P


def _dot(a, b):
    return lax.dot_general(a, b, (((1,), (0,)), ((), ())),
                           preferred_element_type=jnp.float32, precision=PREC)


def _dot_fast(a, b):
    return lax.dot_general(a.astype(jnp.bfloat16), b.astype(jnp.bfloat16),
                           (((1,), (0,)), ((), ())),
                           preferred_element_type=jnp.float32)


def _dot_x(a, b):
    return lax.dot_general(a, b, (((1,), (0,)), ((), ())),
                           preferred_element_type=jnp.float32,
                           precision=jax.lax.Precision.HIGHEST)


def _ln(x, scale, bias):
    m = jnp.mean(x, axis=-1, keepdims=True)
    d = x - m
    v = jnp.mean(d * d, axis=-1, keepdims=True)
    return d * lax.rsqrt(v + 1e-5) * scale + bias


def _erf(x):
    ax = jnp.abs(x)
    t = 1.0 / (1.0 + 0.3275911 * ax)
    y = ((((1.061405429 * t - 1.453152027) * t + 1.421413741) * t
          - 0.284496736) * t + 0.254829592) * t
    y = 1.0 - y * jnp.exp(-ax * ax)
    return jnp.sign(x) * y


def _gelu(x):
    return 0.5 * x * (1.0 + _erf(x * (1.0 / math.sqrt(2.0))))


_NW = 32
_B_PER_W = SEQ // _NW


def _emb_gather(emb, idx):
    mesh = plsc.VectorSubcoreMesh(core_axis_name="c", subcore_axis_name="s")

    @functools.partial(
        pl.kernel, mesh=mesh,
        out_type=jax.ShapeDtypeStruct((SEQ, H_DIM), jnp.float32),
        scratch_types=[
            pltpu.VMEM((_B_PER_W,), jnp.int32),
            pltpu.VMEM((_B_PER_W, H_DIM), jnp.float32),
            pltpu.SemaphoreType.DMA,
        ],
    )
    def k(emb_hbm, idx_hbm, out_hbm, idx_v, rows_v, sem):
        wid = lax.axis_index("s") * 2 + lax.axis_index("c")
        base = wid * _B_PER_W
        pltpu.sync_copy(idx_hbm.at[pl.ds(base, _B_PER_W)], idx_v)
        pltpu.async_copy(emb_hbm.at[idx_v], rows_v, sem).wait()
        pltpu.sync_copy(rows_v, out_hbm.at[pl.ds(base, _B_PER_W)])

    return k(emb, idx)


_AB = 512
_NAB = SEQ // _AB


def _attn1_body(h_ref, n1s_ref, n1b_ref, wdkv_ref, wdq_ref, wuk_ref,
                wuv_ref, wuq_ref, wkr_ref, wqr_ref, cos_ref, sin_ref,
                p_ref, kc_ref, kr_ref, qc_ref, qr_ref, v_ref):
    h1 = _ln(h_ref[...], n1s_ref[...], n1b_ref[...])
    c_kv = _dot(h1, wdkv_ref[...])
    c_q = _dot(h1, wdq_ref[...])
    qc_ref[...] = _dot(c_q, wuq_ref[...])
    kc_ref[...] = _dot(c_kv, wuk_ref[...])
    v_ref[...] = _dot(c_kv, wuv_ref[...])
    k_r = _dot(h1, wkr_ref[...])
    q_r = _dot(c_q, wqr_ref[...])
    cos = cos_ref[...]
    sin = sin_ref[...]
    pm = p_ref[...]
    kr_ref[...] = k_r * cos + _dot_x(k_r, pm) * sin
    qr_ref[...] = q_r * cos + _dot_x(q_r, pm) * sin


def _attn2_body(kc_ref, kr_ref, qc_ref, qr_ref, v_ref, h_ref, wo_ref,
                out_ref):
    kc = kc_ref[...]
    kr = kr_ref[...]
    qc = qc_ref[...]
    qr = qr_ref[...]
    v = v_ref[...]
    outs = []
    for hh in range(N_HEADS):
        s32 = slice(hh * D_ROPE, (hh + 1) * D_ROPE)
        s64 = slice(hh * D_HEAD, (hh + 1) * D_HEAD)
        k_cat = jnp.concatenate([kc[:, s32], kr[:, s32]], axis=1)
        q_cat = jnp.concatenate([qc[:, s32], qr[:, s32]], axis=1)
        s = lax.dot_general(k_cat, q_cat, (((1,), (1,)), ((), ())),
                            preferred_element_type=jnp.float32,
                            precision=PREC) * (1.0 / math.sqrt(D_HEAD))
        outs.append(_dot(s, v[:, s64]))
    attn = jnp.concatenate(outs, axis=1)
    out_ref[...] = _dot(attn, wo_ref[...]) + h_ref[...]


def _attn_block(h, n1s, n1b, wdkv, wdq, wuk, wuv, wuq, wkr, wqr, wo):
    kc, kr, qc, qr, v = pl.pallas_call(
        _attn1_body,
        grid=(_NAB,),
        in_specs=[
            pl.BlockSpec((_AB, H_DIM), lambda r: (r, 0)),
            pl.BlockSpec((1, H_DIM), lambda r: (0, 0)),
            pl.BlockSpec((1, H_DIM), lambda r: (0, 0)),
            pl.BlockSpec((H_DIM, C_DIM), lambda r: (0, 0)),
            pl.BlockSpec((H_DIM, C_DIM), lambda r: (0, 0)),
            pl.BlockSpec((C_DIM, UP_DIM), lambda r: (0, 0)),
            pl.BlockSpec((C_DIM, H_DIM), lambda r: (0, 0)),
            pl.BlockSpec((C_DIM, UP_DIM), lambda r: (0, 0)),
            pl.BlockSpec((H_DIM, UP_DIM), lambda r: (0, 0)),
            pl.BlockSpec((C_DIM, UP_DIM), lambda r: (0, 0)),
            pl.BlockSpec((_AB, UP_DIM), lambda r: (r, 0)),
            pl.BlockSpec((_AB, UP_DIM), lambda r: (r, 0)),
            pl.BlockSpec((UP_DIM, UP_DIM), lambda r: (0, 0)),
        ],
        out_specs=[
            pl.BlockSpec((_AB, UP_DIM), lambda r: (r, 0)),
            pl.BlockSpec((_AB, UP_DIM), lambda r: (r, 0)),
            pl.BlockSpec((_AB, UP_DIM), lambda r: (r, 0)),
            pl.BlockSpec((_AB, UP_DIM), lambda r: (r, 0)),
            pl.BlockSpec((_AB, H_DIM), lambda r: (r, 0)),
        ],
        out_shape=[
            jax.ShapeDtypeStruct((SEQ, UP_DIM), jnp.float32),
            jax.ShapeDtypeStruct((SEQ, UP_DIM), jnp.float32),
            jax.ShapeDtypeStruct((SEQ, UP_DIM), jnp.float32),
            jax.ShapeDtypeStruct((SEQ, UP_DIM), jnp.float32),
            jax.ShapeDtypeStruct((SEQ, H_DIM), jnp.float32),
        ],
    )(h, n1s, n1b, wdkv, wdq, wuk, wuv, wuq, wkr, wqr, COSF, SINF, PMAT)
    return pl.pallas_call(
        _attn2_body,
        grid=(_NAB,),
        in_specs=[
            pl.BlockSpec((_AB, UP_DIM), lambda i: (i, 0)),
            pl.BlockSpec((_AB, UP_DIM), lambda i: (i, 0)),
            pl.BlockSpec((SEQ, UP_DIM), lambda i: (0, 0)),
            pl.BlockSpec((SEQ, UP_DIM), lambda i: (0, 0)),
            pl.BlockSpec((SEQ, H_DIM), lambda i: (0, 0)),
            pl.BlockSpec((_AB, H_DIM), lambda i: (i, 0)),
            pl.BlockSpec((H_DIM, H_DIM), lambda i: (0, 0)),
        ],
        out_specs=pl.BlockSpec((_AB, H_DIM), lambda i: (i, 0)),
        out_shape=jax.ShapeDtypeStruct((SEQ, H_DIM), jnp.float32),
    )(kc, kr, qc, qr, v, h, wo)


_RB = 512
_NRB = SEQ // _RB


def _shared_body(a_ref, h_ref, n2s_ref, n2b_ref, up_ref, dn_ref, rt_ref,
                 y_ref, x_ref, w_ref):
    xx = _ln(a_ref[...], n2s_ref[...], n2b_ref[...])
    g = _gelu(_dot(xx, up_ref[...]))
    y_ref[...] = _dot(g, dn_ref[...]) + xx + h_ref[...]
    x_ref[...] = xx
    logits = _dot(xx, rt_ref[...])
    p = jnp.exp(logits - jnp.max(logits, axis=1, keepdims=True))
    p = p / jnp.sum(p, axis=1, keepdims=True)
    m1 = jnp.max(p, axis=1, keepdims=True)
    p_wo1 = jnp.where(p >= m1, -jnp.inf, p)
    m2 = jnp.max(p_wo1, axis=1, keepdims=True)
    w_ref[...] = jnp.where(p >= m2, p, 0.0)


def _shared_and_router(a, h, n2s, n2b, sh_up, sh_down, router):
    return pl.pallas_call(
        _shared_body,
        grid=(_NRB,),
        in_specs=[
            pl.BlockSpec((_RB, H_DIM), lambda r: (r, 0)),
            pl.BlockSpec((_RB, H_DIM), lambda r: (r, 0)),
            pl.BlockSpec((1, H_DIM), lambda r: (0, 0)),
            pl.BlockSpec((1, H_DIM), lambda r: (0, 0)),
            pl.BlockSpec((H_DIM, E_DIM), lambda r: (0, 0)),
            pl.BlockSpec((E_DIM, H_DIM), lambda r: (0, 0)),
            pl.BlockSpec((H_DIM, N_ROUTED), lambda r: (0, 0)),
        ],
        out_specs=[
            pl.BlockSpec((_RB, H_DIM), lambda r: (r, 0)),
            pl.BlockSpec((_RB, H_DIM), lambda r: (r, 0)),
            pl.BlockSpec((_RB, N_ROUTED), lambda r: (r, 0)),
        ],
        out_shape=[
            jax.ShapeDtypeStruct((SEQ, H_DIM), jnp.float32),
            jax.ShapeDtypeStruct((SEQ, H_DIM), jnp.float32),
            jax.ShapeDtypeStruct((SEQ, N_ROUTED), jnp.float32),
        ],
    )(a, h, n2s, n2b, sh_up, sh_down, router)


N_PAIRS = TOP_K * SEQ


def _route_meta_body(w_ref, inv1_ref, inv2_ref, wt1_ref, wt2_ref, offs_ref):
    w = w_ref[...]
    abool = w > 0.0
    ai = abool.astype(jnp.int32)
    c = ai
    for k in (1, 2, 4):
        c = c + jnp.pad(c, ((0, 0), (k, 0)))[:, :N_ROUTED]
    first = jnp.logical_and(abool, c == 1)
    second = jnp.logical_and(abool, c == 2)
    m2 = jnp.logical_or(first, second).astype(jnp.int32)
    cnt = jnp.sum(m2, axis=0, keepdims=True)
    oc = cnt
    for k in (1, 2, 4):
        oc = oc + jnp.pad(oc, ((0, 0), (k, 0)))[:, :N_ROUTED]
    offs = oc - cnt
    rk = m2
    k = 1
    while k < SEQ:
        rk = rk + jnp.pad(rk, ((k, 0), (0, 0)))[:SEQ, :]
        k *= 2
    pos = offs + (rk - m2)
    fi = first.astype(jnp.int32)
    si = second.astype(jnp.int32)
    inv1_ref[...] = jnp.sum(pos * fi, axis=1, keepdims=True)
    inv2_ref[...] = jnp.sum(pos * si, axis=1, keepdims=True)
    wt1_ref[...] = jnp.sum(w * fi.astype(jnp.float32), axis=1, keepdims=True)
    wt2_ref[...] = jnp.sum(w * si.astype(jnp.float32), axis=1, keepdims=True)
    offs_ref[...] = offs


def _route_meta(w):
    return pl.pallas_call(
        _route_meta_body,
        out_shape=[
            jax.ShapeDtypeStruct((SEQ, 1), jnp.int32),
            jax.ShapeDtypeStruct((SEQ, 1), jnp.int32),
            jax.ShapeDtypeStruct((SEQ, 1), jnp.float32),
            jax.ShapeDtypeStruct((SEQ, 1), jnp.float32),
            jax.ShapeDtypeStruct((1, N_ROUTED), jnp.int32),
        ],
    )(w)


def _sc_scatter_x(xx, idx3):
    mesh = plsc.VectorSubcoreMesh(core_axis_name="c", subcore_axis_name="s")

    @functools.partial(
        pl.kernel, mesh=mesh,
        out_type=jax.ShapeDtypeStruct((N_PAIRS, H_DIM), jnp.float32),
        scratch_types=[
            pltpu.VMEM((TOP_K, _B_PER_W), jnp.int32),
            pltpu.VMEM((_B_PER_W, H_DIM), jnp.float32),
        ],
    )
    def k(xx_hbm, idx_hbm, out_hbm, idx_v, rows_v):
        wid = lax.axis_index("s") * 2 + lax.axis_index("c")
        base = wid * _B_PER_W
        pltpu.sync_copy(idx_hbm.at[wid], idx_v)
        pltpu.sync_copy(xx_hbm.at[pl.ds(base, _B_PER_W)], rows_v)
        pltpu.sync_copy(rows_v, out_hbm.at[idx_v.at[0]])
        pltpu.sync_copy(rows_v, out_hbm.at[idx_v.at[1]])

    return k(xx, idx3)


def _sc_gather2(os_, idx3):
    mesh = plsc.VectorSubcoreMesh(core_axis_name="c", subcore_axis_name="s")

    @functools.partial(
        pl.kernel, mesh=mesh,
        out_type=[
            jax.ShapeDtypeStruct((SEQ, H_DIM), jnp.float32),
            jax.ShapeDtypeStruct((SEQ, H_DIM), jnp.float32),
        ],
        scratch_types=[
            pltpu.VMEM((TOP_K, _B_PER_W), jnp.int32),
            pltpu.VMEM((_B_PER_W, H_DIM), jnp.float32),
        ],
    )
    def k(os_hbm, idx_hbm, g1_hbm, g2_hbm, idx_v, rows_v):
        wid = lax.axis_index("s") * 2 + lax.axis_index("c")
        base = wid * _B_PER_W
        pltpu.sync_copy(idx_hbm.at[wid], idx_v)
        pltpu.sync_copy(os_hbm.at[idx_v.at[0]], rows_v)
        pltpu.sync_copy(rows_v, g1_hbm.at[pl.ds(base, _B_PER_W)])
        pltpu.sync_copy(os_hbm.at[idx_v.at[1]], rows_v)
        pltpu.sync_copy(rows_v, g2_hbm.at[pl.ds(base, _B_PER_W)])

    return k(os_, idx3)


_TB = 128
_NT = N_PAIRS // _TB
_NWORK = _NT + N_ROUTED - 1


def _etile_body(meta_ref, x_ref, up_ref, dn_ref, out_ref):
    wi = pl.program_id(0)
    tile = meta_ref[0, wi]
    start = meta_ref[2, wi]
    end = meta_ref[3, wi]
    fst = meta_ref[4, wi]
    rows = tile * _TB + lax.broadcasted_iota(jnp.int32, (_TB, 1), 0)
    m = jnp.logical_and(rows >= start, rows < end)
    xm = jnp.where(m, x_ref[...], 0.0)
    t = _dot(_gelu(_dot(xm, up_ref[0])), dn_ref[0])

    @pl.when(fst == 1)
    def _():
        out_ref[...] = t

    @pl.when(fst == 0)
    def _():
        out_ref[...] = out_ref[...] + t


def _expert_tiles(meta, xs, r_up, r_down):
    grid_spec = pltpu.PrefetchScalarGridSpec(
        num_scalar_prefetch=1,
        grid=(_NWORK,),
        in_specs=[
            pl.BlockSpec((_TB, H_DIM), lambda wi, m: (m[0, wi], 0)),
            pl.BlockSpec((1, H_DIM, E_DIM), lambda wi, m: (m[1, wi], 0, 0)),
            pl.BlockSpec((1, E_DIM, H_DIM), lambda wi, m: (m[1, wi], 0, 0)),
        ],
        out_specs=pl.BlockSpec((_TB, H_DIM), lambda wi, m: (m[0, wi], 0)),
    )
    return pl.pallas_call(
        _etile_body,
        grid_spec=grid_spec,
        out_shape=jax.ShapeDtypeStruct((N_PAIRS, H_DIM), jnp.float32),
    )(meta, xs, r_up, r_down)


def _combine_body(y_ref, g1_ref, g2_ref, wt1_ref, wt2_ref, nfs_ref, nfb_ref,
                  out_ref):
    routed = g1_ref[...] * wt1_ref[...] + g2_ref[...] * wt2_ref[...]
    out_ref[...] = _ln(routed + y_ref[...], nfs_ref[...], nfb_ref[...])


def _combine_final_ln(y, g1, g2, wt1, wt2, nfs, nfb):
    return pl.pallas_call(
        _combine_body,
        grid=(_NRB,),
        in_specs=[
            pl.BlockSpec((_RB, H_DIM), lambda r: (r, 0)),
            pl.BlockSpec((_RB, H_DIM), lambda r: (r, 0)),
            pl.BlockSpec((_RB, H_DIM), lambda r: (r, 0)),
            pl.BlockSpec((_RB, 1), lambda r: (r, 0)),
            pl.BlockSpec((_RB, 1), lambda r: (r, 0)),
            pl.BlockSpec((1, H_DIM), lambda r: (0, 0)),
            pl.BlockSpec((1, H_DIM), lambda r: (0, 0)),
        ],
        out_specs=pl.BlockSpec((_RB, H_DIM), lambda r: (r, 0)),
        out_shape=jax.ShapeDtypeStruct((SEQ, H_DIM), jnp.float32),
    )(y, g1, g2, wt1, wt2, nfs, nfb)


_CB = 3200
_NCB = N_TOKENS // _CB


def _head_body(mf_ref, proj_ref, out_ref):
    out_ref[...] = _dot(mf_ref[...], proj_ref[...])


_HRB = 1024
_NHRB = SEQ // _HRB


def _lm_head(mf, proj):
    return pl.pallas_call(
        _head_body,
        grid=(_NCB, _NHRB),
        in_specs=[
            pl.BlockSpec((_HRB, H_DIM), lambda c, r: (r, 0)),
            pl.BlockSpec((H_DIM, _CB), lambda c, r: (0, c)),
        ],
        out_specs=pl.BlockSpec((_HRB, _CB), lambda c, r: (r, c)),
        out_shape=jax.ShapeDtypeStruct((SEQ, N_TOKENS), jnp.float32),
    )(mf, proj)


def kernel(x, emb, n1_scale, n1_bias, W_dkv, W_dq, W_uk, W_uv, W_uq, W_kr,
           W_qr, W_o, n2_scale, n2_bias, sh_up, sh_down, r_up, r_down,
           router, nf_scale, nf_bias, proj):
    idx = x.reshape(SEQ)
    h = _emb_gather(emb, idx)
    n1s = n1_scale.reshape(1, H_DIM)
    n1b = n1_bias.reshape(1, H_DIM)
    a = _attn_block(h, n1s, n1b, W_dkv, W_dq, W_uk, W_uv, W_uq, W_kr, W_qr,
                    W_o)
    y, xx, w = _shared_and_router(a, h, n2_scale.reshape(1, H_DIM),
                                  n2_bias.reshape(1, H_DIM), sh_up, sh_down,
                                  router)
    inv1, inv2, wt1, wt2, offs = _route_meta(w)
    offs9 = jnp.concatenate([offs.reshape(N_ROUTED),
                             jnp.array([N_PAIRS], jnp.int32)])
    tau0 = jnp.arange(_NT, dtype=jnp.int32) * _TB
    flag = jnp.logical_and(offs9[None, :N_ROUTED] < (tau0 + _TB)[:, None],
                           offs9[None, 1:] > tau0[:, None])
    flat = flag.reshape(_NT * N_ROUTED)
    order = jnp.argsort(jnp.logical_not(flat), stable=True).astype(jnp.int32)
    sel = order[:_NWORK]
    nact = jnp.sum(flat.astype(jnp.int32))
    valid = jnp.arange(_NWORK, dtype=jnp.int32) < nact
    tile_w = jnp.where(valid, sel // N_ROUTED, _NT - 1)
    e_w = jnp.where(valid, sel % N_ROUTED, N_ROUTED - 1)
    s_w = jnp.where(valid, jnp.take(offs9, e_w), 0)
    en_w = jnp.where(valid, jnp.take(offs9, e_w + 1), 0)
    fst_w = jnp.where(
        valid,
        jnp.concatenate([jnp.array([1], jnp.int32),
                         (tile_w[1:] != tile_w[:-1]).astype(jnp.int32)]),
        0)
    meta = jnp.stack([tile_w, e_w, s_w, en_w, fst_w]).astype(jnp.int32)
    idx3 = jnp.stack([inv1.reshape(_NW, _B_PER_W),
                      inv2.reshape(_NW, _B_PER_W)], axis=1)
    xs = _sc_scatter_x(xx, idx3)
    os_ = _expert_tiles(meta, xs, r_up, r_down)
    g1, g2 = _sc_gather2(os_, idx3)
    mf = _combine_final_ln(y, g1, g2, wt1, wt2, nf_scale.reshape(1, H_DIM),
                           nf_bias.reshape(1, H_DIM))
    logits = _lm_head(mf, proj)
    return logits[None]
